# baseline (speedup 1.0000x reference)
import functools

import numpy as np
import jax
import jax.numpy as jnp
from jax import lax
from jax.experimental import pallas as pl
from jax.experimental.pallas import tpu as pltpu
from jax.experimental.pallas import tpu_sc as plsc

_LN2 = float(np.log(2.0))

E_TOTAL = 320000
N_NODES = 10000
_NW = 32
_PER_W = E_TOTAL // _NW
_GRP = 80
_GPW = _PER_W // _GRP
_CGRP = 5
_NCH = _GPW // _CGRP
_CH_E = _GRP * _CGRP
N_PAD = 10240
_NPT = N_PAD // 16
_TE = 2000


def _sp2(x):
    return jnp.maximum(x, 0.0) + jnp.log(1.0 + jnp.exp(-jnp.abs(x))) - _LN2


def _sigmoid(x):
    return 1.0 / (1.0 + jnp.exp(-x))


def _full(shape):
    return pl.BlockSpec(shape, lambda i: tuple(0 for _ in shape))


def _bdot(a, b):
    return jnp.dot(a, b, preferred_element_type=jnp.float32)


def _edge_encoder(edge_feat, W0, b0, W1, b1):
    E, D = edge_feat.shape

    def body(x_ref, w0, b0r, w1, b1r, o_ref):
        h = _sp2(_bdot(x_ref[...], w0[...]) + b0r[...])
        o_ref[...] = _sp2(_bdot(h.astype(jnp.bfloat16), w1[...]) + b1r[...])

    return pl.pallas_call(
        body,
        grid=(E // _TE,),
        in_specs=[pl.BlockSpec((_TE, D), lambda i: (i, 0)),
                  _full(W0.shape), _full((1, 64)),
                  _full(W1.shape), _full((1, 32))],
        out_specs=pl.BlockSpec((_TE, 32), lambda i: (i, 0)),
        out_shape=jax.ShapeDtypeStruct((E, 32), jnp.float32),
    )(edge_feat.astype(jnp.bfloat16), W0.astype(jnp.bfloat16),
      b0.reshape(1, -1), W1.astype(jnp.bfloat16), b1.reshape(1, -1))


def _node_state_encoder(nf2, emb, nW0, nb0, nW1, nb1, st, sW0, sb0, sW1, sb1):
    ntypes = emb.shape[0]

    def body(nf_ref, emb_ref, nw0, nb0r, nw1, nb1r, st_ref, sw0, sb0r, sw1,
             sb1r, v_ref, u_ref):
        ids = nf_ref[...]
        oh = (ids == lax.broadcasted_iota(jnp.int32, (1, ntypes), 1)
              ).astype(jnp.float32)
        v = oh @ emb_ref[...]
        v = _sp2(v @ nw0[...] + nb0r[...])
        v_ref[...] = _sp2(v @ nw1[...] + nb1r[...])
        u = _sp2(st_ref[...] @ sw0[...] + sb0r[...])
        u_ref[...] = _sp2(u @ sw1[...] + sb1r[...])

    return pl.pallas_call(
        body,
        out_shape=[jax.ShapeDtypeStruct((N_NODES, 32), jnp.float32),
                   jax.ShapeDtypeStruct((1, 32), jnp.float32)],
    )(nf2, emb, nW0, nb0.reshape(1, -1), nW1, nb1.reshape(1, -1),
      st, sW0, sb0.reshape(1, -1), sW1, sb1.reshape(1, -1))


def _bias_eff(u, W1_u, b1):
    def body(u_ref, wu, b1r, be_ref):
        be_ref[...] = u_ref[...] @ wu[...] + b1r[...]

    return pl.pallas_call(
        body, out_shape=jax.ShapeDtypeStruct((1, 64), jnp.float32),
    )(u, W1_u, b1.reshape(1, -1))


def _node_prep(v_in, u_in, node_func, state_func, W1_u, b1):
    (fW0, fb0), (fW1, fb1) = node_func
    (gW0, gb0), (gW1, gb1) = state_func

    def body(v_ref, u_ref, fw0, fb0r, fw1, fb1r, gw0, gb0r, gw1, gb1r, wu,
             b1r, vp_ref, up_ref, be_ref):
        vp = _sp2(v_ref[...] @ fw0[...] + fb0r[...])
        vp_ref[...] = _sp2(vp @ fw1[...] + fb1r[...])
        up = _sp2(u_ref[...] @ gw0[...] + gb0r[...])
        up = _sp2(up @ gw1[...] + gb1r[...])
        up_ref[...] = up
        be_ref[...] = up @ wu[...] + b1r[...]

    return pl.pallas_call(
        body,
        out_shape=[jax.ShapeDtypeStruct((N_NODES, 32), jnp.float32),
                   jax.ShapeDtypeStruct((1, 32), jnp.float32),
                   jax.ShapeDtypeStruct((1, 64), jnp.float32)],
    )(v_in, u_in, fW0, fb0.reshape(1, -1), fW1, fb1.reshape(1, -1),
      gW0, gb0.reshape(1, -1), gW1, gb1.reshape(1, -1), W1_u,
      b1.reshape(1, -1))


def _edge_update(e_in, vs, vd, edge_func, W1_vs, W1_vd, W1_e, b1_eff,
                 W2, b2, W3, b3):
    has_ef = edge_func is not None
    ngrid = E_TOTAL // _TE
    bf = jnp.bfloat16
    W1cat = jnp.concatenate([W1_vs, W1_vd, W1_e], axis=0).astype(bf)

    def body(e_ref, vs_ref, vd_ref, *rest):
        if has_ef:
            (fw0, fb0r, fw1, fb1r, w1c, be, w2, b2r, w3, b3r,
             enew_ref, eout_ref, sume_ref) = rest
        else:
            (w1c, be, w2, b2r, w3, b3r,
             enew_ref, eout_ref, sume_ref) = rest
        i = pl.program_id(0)
        e_t = e_ref[...]
        if has_ef:
            ep = _sp2(_bdot(e_t.astype(bf), fw0[...]) + fb0r[...])
            ep = _sp2(_bdot(ep.astype(bf), fw1[...]) + fb1r[...])
        else:
            ep = e_t
        x1 = jnp.concatenate([vs_ref[...], vd_ref[...], ep.astype(bf)],
                             axis=1)
        h = _sp2(_bdot(x1, w1c[...]) + be[...])
        h = _sp2(_bdot(h.astype(bf), w2[...]) + b2r[...])
        enew = _sp2(_bdot(h.astype(bf), w3[...]) + b3r[...])
        enew_ref[...] = enew
        eout_ref[...] = enew + e_t

        @pl.when(i == 0)
        def _():
            sume_ref[...] = jnp.zeros_like(sume_ref)

        sume_ref[...] += jnp.sum(enew, axis=0, keepdims=True)

    tile = pl.BlockSpec((_TE, 32), lambda i: (i, 0))
    w_in = ([_full((32, 64)), _full((1, 64)), _full((64, 32)), _full((1, 32))]
            if has_ef else [])
    w_in += [_full((96, 64)), _full((1, 64)),
             _full((64, 64)), _full((1, 64)), _full((64, 32)), _full((1, 32))]
    args = [e_in, vs, vd]
    if has_ef:
        (fW0, fb0), (fW1, fb1) = edge_func
        args += [fW0.astype(bf), fb0.reshape(1, -1),
                 fW1.astype(bf), fb1.reshape(1, -1)]
    args += [W1cat, b1_eff, W2.astype(bf), b2.reshape(1, -1),
             W3.astype(bf), b3.reshape(1, -1)]
    return pl.pallas_call(
        body,
        grid=(ngrid,),
        in_specs=[tile, tile, tile] + w_in,
        out_specs=[tile, tile, _full((1, 32))],
        out_shape=[jax.ShapeDtypeStruct((E_TOTAL, 32), jnp.float32),
                   jax.ShapeDtypeStruct((E_TOTAL, 32), jnp.float32),
                   jax.ShapeDtypeStruct((1, 32), jnp.float32)],
    )(*args)


def _node_update(v_in, vp, S, deg2, up, sume, u_in, conv_node, conv_state):
    (cW1, cb1), (cW2, cb2), (cW3, cb3) = conv_node
    (sW1, sb1), (sW2, sb2), (sW3, sb3) = conv_state

    def body(vin_ref, vp_ref, s_ref, d_ref, up_ref, sume_ref, uin_ref,
             cwa, cwb, cwc, cb1r, cw2, cb2r, cw3, cb3r,
             swa, swb, swc, sb1r, sw2, sb2r, sw3, sb3r,
             vout_ref, uout_ref):
        deg = (d_ref[0][0:N_NODES, 0:1] + d_ref[1][0:N_NODES, 0:1])
        ve = ((s_ref[0][0:N_NODES, :] + s_ref[1][0:N_NODES, :])
              / jnp.maximum(deg, 1.0))
        up_v = up_ref[...]
        h = _sp2(vp_ref[...] @ cwa[...] + ve @ cwb[...]
                 + up_v @ cwc[...] + cb1r[...])
        h = _sp2(h @ cw2[...] + cb2r[...])
        vnew = _sp2(h @ cw3[...] + cb3r[...])
        vout_ref[...] = vnew + vin_ref[...]
        mean_v = jnp.mean(vnew, axis=0, keepdims=True)
        mean_e = sume_ref[...] * (1.0 / E_TOTAL)
        s = _sp2(up_v @ swa[...] + mean_v @ swb[...] + mean_e @ swc[...]
                 + sb1r[...])
        s = _sp2(s @ sw2[...] + sb2r[...])
        unew = _sp2(s @ sw3[...] + sb3r[...])
        uout_ref[...] = unew + uin_ref[...]

    return pl.pallas_call(
        body,
        out_shape=[jax.ShapeDtypeStruct((N_NODES, 32), jnp.float32),
                   jax.ShapeDtypeStruct((1, 32), jnp.float32)],
    )(v_in, vp, S, deg2, up, sume, u_in,
      cW1[0:32], cW1[32:64], cW1[64:96], cb1.reshape(1, -1),
      cW2, cb2.reshape(1, -1), cW3, cb3.reshape(1, -1),
      sW1[0:32], sW1[32:64], sW1[64:96], sb1.reshape(1, -1),
      sW2, sb2.reshape(1, -1), sW3, sb3.reshape(1, -1))


def _lstm_step(q, h, c, wih, whh, bihr, bhhr):
    g = q @ wih[...] + bihr[...] + h @ whh[...] + bhhr[...]
    i_ = _sigmoid(g[:, 0:32])
    f_ = _sigmoid(g[:, 32:64])
    gg = jnp.tanh(g[:, 64:96])
    o_ = _sigmoid(g[:, 96:128])
    c = f_ * c + i_ * gg
    h = o_ * jnp.tanh(c)
    return h, c


def _node_s2s(feat, wihT, whhT, bih, bhh):
    def body(feat_ref, wih, whh, bihr, bhhr, q_ref):
        feat_v = feat_ref[...]
        h = jnp.zeros((1, 32), jnp.float32)
        c = jnp.zeros((1, 32), jnp.float32)
        q = jnp.zeros((1, 64), jnp.float32)
        for _ in range(2):
            h, c = _lstm_step(q, h, c, wih, whh, bihr, bhhr)
            logits = jnp.sum(feat_v * h, axis=1, keepdims=True)
            m = jnp.max(logits, axis=0, keepdims=True)
            a = jnp.exp(logits - m)
            z = jnp.sum(a, axis=0, keepdims=True)
            r = jnp.sum((a / z) * feat_v, axis=0, keepdims=True)
            q = jnp.concatenate([h, r], axis=1)
        q_ref[...] = q

    return pl.pallas_call(
        body, out_shape=jax.ShapeDtypeStruct((1, 64), jnp.float32),
    )(feat, wihT, whhT, bih, bhh)


def _edge_s2s_pass(feat, wihT, whhT, bih, bhh, prev):
    first = prev is None
    ngrid = E_TOTAL // _TE

    def body(feat_ref, wih, whh, bihr, bhhr, *rest):
        if first:
            m_ref, z_ref, r_ref, h_ref, c_ref = rest
        else:
            m1, z1, r1, h1, c1, m_ref, z_ref, r_ref, h_ref = rest
        i = pl.program_id(0)

        @pl.when(i == 0)
        def _():
            if first:
                q = jnp.zeros((1, 64), jnp.float32)
                hp = jnp.zeros((1, 32), jnp.float32)
                cp = jnp.zeros((1, 32), jnp.float32)
            else:
                q = jnp.concatenate([h1[...], r1[...] / z1[...]], axis=1)
                hp = h1[...]
                cp = c1[...]
            h, c = _lstm_step(q, hp, cp, wih, whh, bihr, bhhr)
            h_ref[...] = h
            if first:
                c_ref[...] = c
            m_ref[...] = jnp.full((1, 1), -1e30, jnp.float32)
            z_ref[...] = jnp.zeros((1, 1), jnp.float32)
            r_ref[...] = jnp.zeros((1, 32), jnp.float32)

        h = h_ref[...]
        feat_t = feat_ref[...]
        logits = jnp.sum(feat_t * h, axis=1, keepdims=True)
        mt = jnp.max(logits, axis=0, keepdims=True)
        m_old = m_ref[...]
        m_new = jnp.maximum(m_old, mt)
        sc = jnp.exp(m_old - m_new)
        a = jnp.exp(logits - m_new)
        z_ref[...] = z_ref[...] * sc + jnp.sum(a, axis=0, keepdims=True)
        r_ref[...] = r_ref[...] * sc + jnp.sum(a * feat_t, axis=0,
                                               keepdims=True)
        m_ref[...] = m_new

    tile = pl.BlockSpec((_TE, 32), lambda i: (i, 0))
    small = [_full((1, 1)), _full((1, 1)), _full((1, 32)), _full((1, 32))]
    in_specs = [tile, _full((64, 128)), _full((32, 128)), _full((1, 128)),
                _full((1, 128))]
    out_shape = [jax.ShapeDtypeStruct((1, 1), jnp.float32),
                 jax.ShapeDtypeStruct((1, 1), jnp.float32),
                 jax.ShapeDtypeStruct((1, 32), jnp.float32),
                 jax.ShapeDtypeStruct((1, 32), jnp.float32)]
    out_specs = small[:]
    args = [feat, wihT, whhT, bih, bhh]
    if first:
        out_shape.append(jax.ShapeDtypeStruct((1, 32), jnp.float32))
        out_specs.append(_full((1, 32)))
    else:
        in_specs += small + [_full((1, 32))]
        args += list(prev)
    return pl.pallas_call(
        body, grid=(ngrid,), in_specs=in_specs, out_specs=out_specs,
        out_shape=out_shape,
    )(*args)


_MESH = dict(core_axis_name="c", subcore_axis_name="s")


def _sc_gather(vtab, src2, dst2):
    mesh = plsc.VectorSubcoreMesh(**_MESH)

    @functools.partial(
        pl.kernel, mesh=mesh,
        compiler_params=pltpu.CompilerParams(use_tc_tiling_on_sc=False),
        out_type=[jax.ShapeDtypeStruct((E_TOTAL, 32), jnp.bfloat16),
                  jax.ShapeDtypeStruct((E_TOTAL, 32), jnp.bfloat16)],
        scratch_types=[pltpu.VMEM((_GPW, _GRP), jnp.int32),
                       pltpu.VMEM((_GPW, _GRP), jnp.int32),
                       pltpu.VMEM((_CH_E, 32), jnp.bfloat16),
                       pltpu.VMEM((_CH_E, 32), jnp.bfloat16),
                       pltpu.SemaphoreType.DMA],
    )
    def k(vtab_ref, src_ref, dst_ref, vs_ref, vd_ref, idx_s, idx_d,
          buf_s, buf_d, sem):
        wid = lax.axis_index("s") * 2 + lax.axis_index("c")
        pltpu.sync_copy(src_ref.at[wid], idx_s)
        pltpu.sync_copy(dst_ref.at[wid], idx_d)
        ebase = wid * _PER_W

        def chunk(ci, carry):
            cps = []
            for j in range(_CGRP):
                g = ci * _CGRP + j
                cps.append(pltpu.async_copy(
                    vtab_ref.at[idx_s.at[g]],
                    buf_s.at[pl.ds(j * _GRP, _GRP)], sem))
                cps.append(pltpu.async_copy(
                    vtab_ref.at[idx_d.at[g]],
                    buf_d.at[pl.ds(j * _GRP, _GRP)], sem))
            for cp in cps:
                cp.wait()
            ob = ebase + ci * _CH_E
            pltpu.sync_copy(buf_s, vs_ref.at[pl.ds(ob, _CH_E)])
            pltpu.sync_copy(buf_d, vd_ref.at[pl.ds(ob, _CH_E)])
            return carry

        lax.fori_loop(0, _NCH, chunk, 0)

    return k(vtab, src2, dst2)


def _sc_scatter_add(enew, dst2, zeros32):
    mesh = plsc.VectorSubcoreMesh(**_MESH)

    @functools.partial(
        pl.kernel, mesh=mesh,
        compiler_params=pltpu.CompilerParams(use_tc_tiling_on_sc=False),
        out_type=jax.ShapeDtypeStruct((2 * N_PAD, 32), jnp.float32),
        scratch_types=[pltpu.VMEM((_GPW, _GRP), jnp.int32),
                       pltpu.VMEM((_CH_E, 32), jnp.float32),
                       pltpu.VMEM_SHARED((N_PAD, 32), jnp.float32),
                       pltpu.SemaphoreType.DMA],
    )
    def k(enew_ref, dst_ref, z_ref, out_ref, idx_d, rows, accum, sem):
        cid = lax.axis_index("c")
        sid = lax.axis_index("s")
        wid = sid * 2 + cid
        pltpu.sync_copy(dst_ref.at[wid], idx_d)
        pltpu.sync_copy(z_ref.at[pl.ds(sid * _NPT, _NPT)],
                        accum.at[pl.ds(sid * _NPT, _NPT)])
        plsc.subcore_barrier()
        ebase = wid * _PER_W

        def chunk(ci, carry):
            pltpu.sync_copy(enew_ref.at[pl.ds(ebase + ci * _CH_E, _CH_E)],
                            rows)
            cps = [pltpu.async_copy(rows.at[pl.ds(j * _GRP, _GRP)],
                                    accum.at[idx_d.at[ci * _CGRP + j]],
                                    sem, add=True)
                   for j in range(_CGRP)]
            for cp in cps:
                cp.wait()
            return carry

        lax.fori_loop(0, _NCH, chunk, 0)
        plsc.subcore_barrier()
        pltpu.sync_copy(accum.at[pl.ds(sid * _NPT, _NPT)],
                        out_ref.at[pl.ds(cid * N_PAD + sid * _NPT, _NPT)])

    return k(enew, dst2, zeros32).reshape(2, N_PAD, 32)


def _sc_degree(dst2, ones16, zeros16):
    mesh = plsc.VectorSubcoreMesh(**_MESH)

    @functools.partial(
        pl.kernel, mesh=mesh,
        compiler_params=pltpu.CompilerParams(use_tc_tiling_on_sc=False),
        out_type=jax.ShapeDtypeStruct((2 * N_PAD, 16), jnp.float32),
        scratch_types=[pltpu.VMEM((_GPW, _GRP), jnp.int32),
                       pltpu.VMEM((_GRP, 16), jnp.float32),
                       pltpu.VMEM_SHARED((N_PAD, 16), jnp.float32),
                       pltpu.SemaphoreType.DMA],
    )
    def k(dst_ref, ones_ref, z_ref, out_ref, idx_d, onesbuf, accum, sem):
        cid = lax.axis_index("c")
        sid = lax.axis_index("s")
        wid = sid * 2 + cid
        pltpu.sync_copy(dst_ref.at[wid], idx_d)
        pltpu.sync_copy(ones_ref, onesbuf)
        pltpu.sync_copy(z_ref.at[pl.ds(sid * _NPT, _NPT)],
                        accum.at[pl.ds(sid * _NPT, _NPT)])
        plsc.subcore_barrier()

        def chunk(ci, carry):
            cps = [pltpu.async_copy(onesbuf,
                                    accum.at[idx_d.at[ci * _CGRP + j]],
                                    sem, add=True)
                   for j in range(_CGRP)]
            for cp in cps:
                cp.wait()
            return carry

        lax.fori_loop(0, _NCH, chunk, 0)
        plsc.subcore_barrier()
        pltpu.sync_copy(accum.at[pl.ds(sid * _NPT, _NPT)],
                        out_ref.at[pl.ds(cid * N_PAD + sid * _NPT, _NPT)])

    return k(dst2, ones16, zeros16).reshape(2, N_PAD, 16)


def kernel(edge_index, edge_feat, node_feat, state_feat, params):
    src2 = edge_index[0].reshape(_NW, _GPW, _GRP)
    dst2 = edge_index[1].reshape(_NW, _GPW, _GRP)
    nf2 = node_feat.reshape(N_NODES, 1)
    zeros32 = jnp.zeros((N_PAD, 32), jnp.float32)
    zeros16 = jnp.zeros((N_PAD, 16), jnp.float32)
    ones16 = jnp.ones((_GRP, 16), jnp.float32)

    p = params
    (eW0, eb0), (eW1, eb1) = p['edge_enc']
    e = _edge_encoder(edge_feat, eW0, eb0, eW1, eb1)
    (nW0, nb0), (nW1, nb1) = p['node_enc']
    (sW0, sb0), (sW1, sb1) = p['state_enc']
    v, u = _node_state_encoder(nf2, p['node_emb'], nW0, nb0, nW1, nb1,
                               state_feat, sW0, sb0, sW1, sb1)

    deg2 = _sc_degree(dst2, ones16, zeros16)

    for b, bp in enumerate(p['blocks']):
        W1, b1 = bp['conv_edge'][0]
        W1_vs, W1_vd, W1_e, W1_u = W1[0:32], W1[32:64], W1[64:96], W1[96:128]
        if b == 0:
            vp, up = v, u
            b1_eff = _bias_eff(u, W1_u, b1)
        else:
            vp, up, b1_eff = _node_prep(v, u, bp['node_func'],
                                        bp['state_func'], W1_u, b1)
        vs_g, vd_g = _sc_gather(vp.astype(jnp.bfloat16), src2, dst2)
        (W2, b2), (W3, b3) = bp['conv_edge'][1], bp['conv_edge'][2]
        enew, eout, sume = _edge_update(
            e, vs_g, vd_g, bp['edge_func'] if b > 0 else None,
            W1_vs, W1_vd, W1_e, b1_eff, W2, b2, W3, b3)
        S = _sc_scatter_add(enew, dst2, zeros32)
        v, u = _node_update(v, vp, S, deg2, up, sume, u,
                            bp['conv_node'], bp['conv_state'])
        e = eout

    ns = p['node_s2s']
    q_n = _node_s2s(v, ns['W_ih'].T, ns['W_hh'].T,
                    ns['b_ih'].reshape(1, -1), ns['b_hh'].reshape(1, -1))
    es = p['edge_s2s']
    wihT, whhT = es['W_ih'].T, es['W_hh'].T
    bih, bhh = es['b_ih'].reshape(1, -1), es['b_hh'].reshape(1, -1)
    m1, z1, r1, h1, c1 = _edge_s2s_pass(e, wihT, whhT, bih, bhh, None)
    m2, z2, r2, h2 = _edge_s2s_pass(e, wihT, whhT, bih, bhh,
                                    (m1, z1, r1, h1, c1))
    edge_vec = jnp.concatenate([h2, r2 / z2], axis=1).reshape(64)
    return jnp.hstack([q_n.reshape(64), edge_vec, u.reshape(32)])


# speedup vs baseline: 1.0111x; 1.0111x over previous
import functools

import numpy as np
import jax
import jax.numpy as jnp
from jax import lax
from jax.experimental import pallas as pl
from jax.experimental.pallas import tpu as pltpu
from jax.experimental.pallas import tpu_sc as plsc

_LN2 = float(np.log(2.0))

E_TOTAL = 320000
N_NODES = 10000
_NW = 32
_PER_W = E_TOTAL // _NW
_GRP = 80
_GPW = _PER_W // _GRP
_CGRP = 5
_NCH = _GPW // _CGRP
_CH_E = _GRP * _CGRP
N_PAD = 10240
_NPT = N_PAD // 16
_TE = 2000


def _sp2(x):
    return jnp.maximum(x, 0.0) + jnp.log(1.0 + jnp.exp(-jnp.abs(x))) - _LN2


_LOG2E = float(np.log2(np.e))


def _g2(x):
    return jnp.maximum(x, 0.0) + jnp.log2(1.0 + jnp.exp2(jnp.minimum(x, -x)))


def _sigmoid(x):
    return 1.0 / (1.0 + jnp.exp(-x))


def _full(shape):
    return pl.BlockSpec(shape, lambda i: tuple(0 for _ in shape))


def _bdot(a, b):
    return jnp.dot(a, b, preferred_element_type=jnp.float32)


def _edge_encoder(edge_feat, W0, b0, W1, b1):
    E, D = edge_feat.shape
    bf = jnp.bfloat16
    W0s = (W0 * _LOG2E).astype(bf)
    W1b = W1.astype(bf)
    b0s = (b0 * _LOG2E).reshape(1, -1)
    b1s = (b1 * _LOG2E - jnp.sum(W1b.astype(jnp.float32), axis=0)).reshape(1, -1)

    def body(x_ref, w0, b0r, w1, b1r, o_ref):
        a = _g2(_bdot(x_ref[...].astype(bf), w0[...]) + b0r[...])
        a = _g2(_bdot(a.astype(bf), w1[...]) + b1r[...])
        o_ref[...] = (a - 1.0) * _LN2

    return pl.pallas_call(
        body,
        grid=(E // _TE,),
        in_specs=[pl.BlockSpec((_TE, D), lambda i: (i, 0)),
                  _full(W0.shape), _full((1, 64)),
                  _full(W1.shape), _full((1, 32))],
        out_specs=pl.BlockSpec((_TE, 32), lambda i: (i, 0)),
        out_shape=jax.ShapeDtypeStruct((E, 32), jnp.float32),
    )(edge_feat, W0s, b0s, W1b, b1s)


def _node_state_encoder(nf2, emb, nW0, nb0, nW1, nb1, st, sW0, sb0, sW1, sb1):
    ntypes = emb.shape[0]

    def body(nf_ref, emb_ref, nw0, nb0r, nw1, nb1r, st_ref, sw0, sb0r, sw1,
             sb1r, v_ref, u_ref):
        ids = nf_ref[...]
        oh = (ids == lax.broadcasted_iota(jnp.int32, (1, ntypes), 1)
              ).astype(jnp.float32)
        v = oh @ emb_ref[...]
        v = _sp2(v @ nw0[...] + nb0r[...])
        v_ref[...] = _sp2(v @ nw1[...] + nb1r[...])
        u = _sp2(st_ref[...] @ sw0[...] + sb0r[...])
        u_ref[...] = _sp2(u @ sw1[...] + sb1r[...])

    return pl.pallas_call(
        body,
        out_shape=[jax.ShapeDtypeStruct((N_NODES, 32), jnp.float32),
                   jax.ShapeDtypeStruct((1, 32), jnp.float32)],
    )(nf2, emb, nW0, nb0.reshape(1, -1), nW1, nb1.reshape(1, -1),
      st, sW0, sb0.reshape(1, -1), sW1, sb1.reshape(1, -1))


def _bias_eff(u, W1_u, b1, corr):
    def body(u_ref, wu, b1r, corr_r, be_ref):
        be_ref[...] = ((u_ref[...] @ wu[...] + b1r[...]) * _LOG2E
                       - corr_r[...])

    return pl.pallas_call(
        body, out_shape=jax.ShapeDtypeStruct((1, 64), jnp.float32),
    )(u, W1_u, b1.reshape(1, -1), corr)


def _node_prep(v_in, u_in, node_func, state_func, W1_u, b1, corr):
    (fW0, fb0), (fW1, fb1) = node_func
    (gW0, gb0), (gW1, gb1) = state_func

    def body(v_ref, u_ref, fw0, fb0r, fw1, fb1r, gw0, gb0r, gw1, gb1r, wu,
             b1r, corr_r, vp_ref, up_ref, be_ref):
        vp = _sp2(v_ref[...] @ fw0[...] + fb0r[...])
        vp_ref[...] = _sp2(vp @ fw1[...] + fb1r[...])
        up = _sp2(u_ref[...] @ gw0[...] + gb0r[...])
        up = _sp2(up @ gw1[...] + gb1r[...])
        up_ref[...] = up
        be_ref[...] = ((up @ wu[...] + b1r[...]) * _LOG2E - corr_r[...])

    return pl.pallas_call(
        body,
        out_shape=[jax.ShapeDtypeStruct((N_NODES, 32), jnp.float32),
                   jax.ShapeDtypeStruct((1, 32), jnp.float32),
                   jax.ShapeDtypeStruct((1, 64), jnp.float32)],
    )(v_in, u_in, fW0, fb0.reshape(1, -1), fW1, fb1.reshape(1, -1),
      gW0, gb0.reshape(1, -1), gW1, gb1.reshape(1, -1), W1_u,
      b1.reshape(1, -1), corr)


def _edge_update(e_in, vs, vd, edge_func, W1_vs, W1_vd, W1_e, b1_eff,
                 W2, b2, W3, b3):
    has_ef = edge_func is not None
    ngrid = E_TOTAL // _TE
    bf = jnp.bfloat16
    if has_ef:
        W1cat = jnp.concatenate(
            [W1_vs * _LOG2E, W1_vd * _LOG2E, W1_e], axis=0).astype(bf)
    else:
        W1cat = (jnp.concatenate([W1_vs, W1_vd, W1_e], axis=0)
                 * _LOG2E).astype(bf)
    W2b, W3b = W2.astype(bf), W3.astype(bf)
    b2s = (b2 * _LOG2E
           - jnp.sum(W2b.astype(jnp.float32), axis=0)).reshape(1, -1)
    b3s = (b3 * _LOG2E
           - jnp.sum(W3b.astype(jnp.float32), axis=0)).reshape(1, -1)

    def body(e_ref, vs_ref, vd_ref, *rest):
        if has_ef:
            (fw0, fb0r, fw1, fb1r, w1c, be, w2, b2r, w3, b3r,
             enew_ref, eout_ref, sume_ref) = rest
        else:
            (w1c, be, w2, b2r, w3, b3r,
             enew_ref, eout_ref, sume_ref) = rest
        i = pl.program_id(0)
        e_t = e_ref[...]
        if has_ef:
            a = _g2(_bdot(e_t.astype(bf), fw0[...]) + fb0r[...])
            a = _g2(_bdot(a.astype(bf), fw1[...]) + fb1r[...])
            ep = a
        else:
            ep = e_t
        x1 = jnp.concatenate([vs_ref[...], vd_ref[...], ep.astype(bf)],
                             axis=1)
        a1 = _g2(_bdot(x1, w1c[...]) + be[...])
        a2 = _g2(_bdot(a1.astype(bf), w2[...]) + b2r[...])
        a3 = _g2(_bdot(a2.astype(bf), w3[...]) + b3r[...])
        enew = (a3 - 1.0) * _LN2
        enew_ref[...] = enew
        eout_ref[...] = enew + e_t

        @pl.when(i == 0)
        def _():
            sume_ref[...] = jnp.zeros_like(sume_ref)

        sume_ref[...] += jnp.sum(enew, axis=0, keepdims=True)

    tile = pl.BlockSpec((_TE, 32), lambda i: (i, 0))
    w_in = ([_full((32, 64)), _full((1, 64)), _full((64, 32)), _full((1, 32))]
            if has_ef else [])
    w_in += [_full((96, 64)), _full((1, 64)),
             _full((64, 64)), _full((1, 64)), _full((64, 32)), _full((1, 32))]
    args = [e_in, vs, vd]
    if has_ef:
        (fW0, fb0), (fW1, fb1) = edge_func
        fW1b = fW1.astype(bf)
        args += [(fW0 * _LOG2E).astype(bf), (fb0 * _LOG2E).reshape(1, -1),
                 fW1b,
                 (fb1 * _LOG2E
                  - jnp.sum(fW1b.astype(jnp.float32), axis=0)
                  ).reshape(1, -1)]
    args += [W1cat, b1_eff, W2b, b2s, W3b, b3s]
    return pl.pallas_call(
        body,
        grid=(ngrid,),
        in_specs=[tile, tile, tile] + w_in,
        out_specs=[tile, tile, _full((1, 32))],
        out_shape=[jax.ShapeDtypeStruct((E_TOTAL, 32), jnp.float32),
                   jax.ShapeDtypeStruct((E_TOTAL, 32), jnp.float32),
                   jax.ShapeDtypeStruct((1, 32), jnp.float32)],
    )(*args)


def _node_update(v_in, vp, S, deg2, up, sume, u_in, conv_node, conv_state):
    (cW1, cb1), (cW2, cb2), (cW3, cb3) = conv_node
    (sW1, sb1), (sW2, sb2), (sW3, sb3) = conv_state

    def body(vin_ref, vp_ref, s_ref, d_ref, up_ref, sume_ref, uin_ref,
             cwa, cwb, cwc, cb1r, cw2, cb2r, cw3, cb3r,
             swa, swb, swc, sb1r, sw2, sb2r, sw3, sb3r,
             vout_ref, uout_ref):
        deg = (d_ref[0][0:N_NODES, 0:1] + d_ref[1][0:N_NODES, 0:1])
        ve = ((s_ref[0][0:N_NODES, :] + s_ref[1][0:N_NODES, :])
              / jnp.maximum(deg, 1.0))
        up_v = up_ref[...]
        h = _sp2(vp_ref[...] @ cwa[...] + ve @ cwb[...]
                 + up_v @ cwc[...] + cb1r[...])
        h = _sp2(h @ cw2[...] + cb2r[...])
        vnew = _sp2(h @ cw3[...] + cb3r[...])
        vout_ref[...] = vnew + vin_ref[...]
        mean_v = jnp.mean(vnew, axis=0, keepdims=True)
        mean_e = sume_ref[...] * (1.0 / E_TOTAL)
        s = _sp2(up_v @ swa[...] + mean_v @ swb[...] + mean_e @ swc[...]
                 + sb1r[...])
        s = _sp2(s @ sw2[...] + sb2r[...])
        unew = _sp2(s @ sw3[...] + sb3r[...])
        uout_ref[...] = unew + uin_ref[...]

    return pl.pallas_call(
        body,
        out_shape=[jax.ShapeDtypeStruct((N_NODES, 32), jnp.float32),
                   jax.ShapeDtypeStruct((1, 32), jnp.float32)],
    )(v_in, vp, S, deg2, up, sume, u_in,
      cW1[0:32], cW1[32:64], cW1[64:96], cb1.reshape(1, -1),
      cW2, cb2.reshape(1, -1), cW3, cb3.reshape(1, -1),
      sW1[0:32], sW1[32:64], sW1[64:96], sb1.reshape(1, -1),
      sW2, sb2.reshape(1, -1), sW3, sb3.reshape(1, -1))


def _lstm_step(q, h, c, wih, whh, bihr, bhhr):
    g = q @ wih[...] + bihr[...] + h @ whh[...] + bhhr[...]
    i_ = _sigmoid(g[:, 0:32])
    f_ = _sigmoid(g[:, 32:64])
    gg = jnp.tanh(g[:, 64:96])
    o_ = _sigmoid(g[:, 96:128])
    c = f_ * c + i_ * gg
    h = o_ * jnp.tanh(c)
    return h, c


def _node_s2s(feat, wihT, whhT, bih, bhh):
    def body(feat_ref, wih, whh, bihr, bhhr, q_ref):
        feat_v = feat_ref[...]
        h = jnp.zeros((1, 32), jnp.float32)
        c = jnp.zeros((1, 32), jnp.float32)
        q = jnp.zeros((1, 64), jnp.float32)
        for _ in range(2):
            h, c = _lstm_step(q, h, c, wih, whh, bihr, bhhr)
            logits = jnp.sum(feat_v * h, axis=1, keepdims=True)
            m = jnp.max(logits, axis=0, keepdims=True)
            a = jnp.exp(logits - m)
            z = jnp.sum(a, axis=0, keepdims=True)
            r = jnp.sum((a / z) * feat_v, axis=0, keepdims=True)
            q = jnp.concatenate([h, r], axis=1)
        q_ref[...] = q

    return pl.pallas_call(
        body, out_shape=jax.ShapeDtypeStruct((1, 64), jnp.float32),
    )(feat, wihT, whhT, bih, bhh)


def _edge_s2s_pass(feat, wihT, whhT, bih, bhh, prev):
    first = prev is None
    ngrid = E_TOTAL // _TE

    def body(feat_ref, wih, whh, bihr, bhhr, *rest):
        if first:
            m_ref, z_ref, r_ref, h_ref, c_ref = rest
        else:
            m1, z1, r1, h1, c1, m_ref, z_ref, r_ref, h_ref = rest
        i = pl.program_id(0)

        @pl.when(i == 0)
        def _():
            if first:
                q = jnp.zeros((1, 64), jnp.float32)
                hp = jnp.zeros((1, 32), jnp.float32)
                cp = jnp.zeros((1, 32), jnp.float32)
            else:
                q = jnp.concatenate([h1[...], r1[...] / z1[...]], axis=1)
                hp = h1[...]
                cp = c1[...]
            h, c = _lstm_step(q, hp, cp, wih, whh, bihr, bhhr)
            h_ref[...] = h
            if first:
                c_ref[...] = c
            m_ref[...] = jnp.full((1, 1), -1e30, jnp.float32)
            z_ref[...] = jnp.zeros((1, 1), jnp.float32)
            r_ref[...] = jnp.zeros((1, 32), jnp.float32)

        h = h_ref[...]
        feat_t = feat_ref[...]
        logits = jnp.sum(feat_t * h, axis=1, keepdims=True)
        mt = jnp.max(logits, axis=0, keepdims=True)
        m_old = m_ref[...]
        m_new = jnp.maximum(m_old, mt)
        sc = jnp.exp(m_old - m_new)
        a = jnp.exp(logits - m_new)
        z_ref[...] = z_ref[...] * sc + jnp.sum(a, axis=0, keepdims=True)
        r_ref[...] = r_ref[...] * sc + jnp.sum(a * feat_t, axis=0,
                                               keepdims=True)
        m_ref[...] = m_new

    tile = pl.BlockSpec((_TE, 32), lambda i: (i, 0))
    small = [_full((1, 1)), _full((1, 1)), _full((1, 32)), _full((1, 32))]
    in_specs = [tile, _full((64, 128)), _full((32, 128)), _full((1, 128)),
                _full((1, 128))]
    out_shape = [jax.ShapeDtypeStruct((1, 1), jnp.float32),
                 jax.ShapeDtypeStruct((1, 1), jnp.float32),
                 jax.ShapeDtypeStruct((1, 32), jnp.float32),
                 jax.ShapeDtypeStruct((1, 32), jnp.float32)]
    out_specs = small[:]
    args = [feat, wihT, whhT, bih, bhh]
    if first:
        out_shape.append(jax.ShapeDtypeStruct((1, 32), jnp.float32))
        out_specs.append(_full((1, 32)))
    else:
        in_specs += small + [_full((1, 32))]
        args += list(prev)
    return pl.pallas_call(
        body, grid=(ngrid,), in_specs=in_specs, out_specs=out_specs,
        out_shape=out_shape,
    )(*args)


_MESH = dict(core_axis_name="c", subcore_axis_name="s")


def _sc_gather(vtab, src2, dst2):
    mesh = plsc.VectorSubcoreMesh(**_MESH)

    @functools.partial(
        pl.kernel, mesh=mesh,
        compiler_params=pltpu.CompilerParams(use_tc_tiling_on_sc=False),
        out_type=[jax.ShapeDtypeStruct((E_TOTAL, 32), jnp.bfloat16),
                  jax.ShapeDtypeStruct((E_TOTAL, 32), jnp.bfloat16)],
        scratch_types=[pltpu.VMEM((_GPW, _GRP), jnp.int32),
                       pltpu.VMEM((_GPW, _GRP), jnp.int32),
                       pltpu.VMEM((_CH_E, 32), jnp.bfloat16),
                       pltpu.VMEM((_CH_E, 32), jnp.bfloat16),
                       pltpu.SemaphoreType.DMA],
    )
    def k(vtab_ref, src_ref, dst_ref, vs_ref, vd_ref, idx_s, idx_d,
          buf_s, buf_d, sem):
        wid = lax.axis_index("s") * 2 + lax.axis_index("c")
        pltpu.sync_copy(src_ref.at[wid], idx_s)
        pltpu.sync_copy(dst_ref.at[wid], idx_d)
        ebase = wid * _PER_W

        def chunk(ci, carry):
            cps = []
            for j in range(_CGRP):
                g = ci * _CGRP + j
                cps.append(pltpu.async_copy(
                    vtab_ref.at[idx_s.at[g]],
                    buf_s.at[pl.ds(j * _GRP, _GRP)], sem))
                cps.append(pltpu.async_copy(
                    vtab_ref.at[idx_d.at[g]],
                    buf_d.at[pl.ds(j * _GRP, _GRP)], sem))
            for cp in cps:
                cp.wait()
            ob = ebase + ci * _CH_E
            pltpu.sync_copy(buf_s, vs_ref.at[pl.ds(ob, _CH_E)])
            pltpu.sync_copy(buf_d, vd_ref.at[pl.ds(ob, _CH_E)])
            return carry

        lax.fori_loop(0, _NCH, chunk, 0)

    return k(vtab, src2, dst2)


def _sc_scatter_add(enew, dst2, zeros32):
    mesh = plsc.VectorSubcoreMesh(**_MESH)

    @functools.partial(
        pl.kernel, mesh=mesh,
        compiler_params=pltpu.CompilerParams(use_tc_tiling_on_sc=False),
        out_type=jax.ShapeDtypeStruct((2 * N_PAD, 32), jnp.float32),
        scratch_types=[pltpu.VMEM((_GPW, _GRP), jnp.int32),
                       pltpu.VMEM((_CH_E, 32), jnp.float32),
                       pltpu.VMEM_SHARED((N_PAD, 32), jnp.float32),
                       pltpu.SemaphoreType.DMA],
    )
    def k(enew_ref, dst_ref, z_ref, out_ref, idx_d, rows, accum, sem):
        cid = lax.axis_index("c")
        sid = lax.axis_index("s")
        wid = sid * 2 + cid
        pltpu.sync_copy(dst_ref.at[wid], idx_d)
        pltpu.sync_copy(z_ref.at[pl.ds(sid * _NPT, _NPT)],
                        accum.at[pl.ds(sid * _NPT, _NPT)])
        plsc.subcore_barrier()
        ebase = wid * _PER_W

        def chunk(ci, carry):
            pltpu.sync_copy(enew_ref.at[pl.ds(ebase + ci * _CH_E, _CH_E)],
                            rows)
            cps = [pltpu.async_copy(rows.at[pl.ds(j * _GRP, _GRP)],
                                    accum.at[idx_d.at[ci * _CGRP + j]],
                                    sem, add=True)
                   for j in range(_CGRP)]
            for cp in cps:
                cp.wait()
            return carry

        lax.fori_loop(0, _NCH, chunk, 0)
        plsc.subcore_barrier()
        pltpu.sync_copy(accum.at[pl.ds(sid * _NPT, _NPT)],
                        out_ref.at[pl.ds(cid * N_PAD + sid * _NPT, _NPT)])

    return k(enew, dst2, zeros32).reshape(2, N_PAD, 32)


def _sc_degree(dst2, ones16, zeros16):
    mesh = plsc.VectorSubcoreMesh(**_MESH)

    @functools.partial(
        pl.kernel, mesh=mesh,
        compiler_params=pltpu.CompilerParams(use_tc_tiling_on_sc=False),
        out_type=jax.ShapeDtypeStruct((2 * N_PAD, 16), jnp.float32),
        scratch_types=[pltpu.VMEM((_GPW, _GRP), jnp.int32),
                       pltpu.VMEM((_GRP, 16), jnp.float32),
                       pltpu.VMEM_SHARED((N_PAD, 16), jnp.float32),
                       pltpu.SemaphoreType.DMA],
    )
    def k(dst_ref, ones_ref, z_ref, out_ref, idx_d, onesbuf, accum, sem):
        cid = lax.axis_index("c")
        sid = lax.axis_index("s")
        wid = sid * 2 + cid
        pltpu.sync_copy(dst_ref.at[wid], idx_d)
        pltpu.sync_copy(ones_ref, onesbuf)
        pltpu.sync_copy(z_ref.at[pl.ds(sid * _NPT, _NPT)],
                        accum.at[pl.ds(sid * _NPT, _NPT)])
        plsc.subcore_barrier()

        def chunk(ci, carry):
            cps = [pltpu.async_copy(onesbuf,
                                    accum.at[idx_d.at[ci * _CGRP + j]],
                                    sem, add=True)
                   for j in range(_CGRP)]
            for cp in cps:
                cp.wait()
            return carry

        lax.fori_loop(0, _NCH, chunk, 0)
        plsc.subcore_barrier()
        pltpu.sync_copy(accum.at[pl.ds(sid * _NPT, _NPT)],
                        out_ref.at[pl.ds(cid * N_PAD + sid * _NPT, _NPT)])

    return k(dst2, ones16, zeros16).reshape(2, N_PAD, 16)


def kernel(edge_index, edge_feat, node_feat, state_feat, params):
    src2 = edge_index[0].reshape(_NW, _GPW, _GRP)
    dst2 = edge_index[1].reshape(_NW, _GPW, _GRP)
    nf2 = node_feat.reshape(N_NODES, 1)
    zeros32 = jnp.zeros((N_PAD, 32), jnp.float32)
    zeros16 = jnp.zeros((N_PAD, 16), jnp.float32)
    ones16 = jnp.ones((_GRP, 16), jnp.float32)

    p = params
    (eW0, eb0), (eW1, eb1) = p['edge_enc']
    e = _edge_encoder(edge_feat, eW0, eb0, eW1, eb1)
    (nW0, nb0), (nW1, nb1) = p['node_enc']
    (sW0, sb0), (sW1, sb1) = p['state_enc']
    v, u = _node_state_encoder(nf2, p['node_emb'], nW0, nb0, nW1, nb1,
                               state_feat, sW0, sb0, sW1, sb1)

    deg2 = _sc_degree(dst2, ones16, zeros16)

    for b, bp in enumerate(p['blocks']):
        W1, b1 = bp['conv_edge'][0]
        W1_vs, W1_vd, W1_e, W1_u = W1[0:32], W1[32:64], W1[64:96], W1[96:128]
        corr = (jnp.sum(W1_e.astype(jnp.bfloat16).astype(jnp.float32),
                        axis=0).reshape(1, -1) if b > 0
                else jnp.zeros((1, 64), jnp.float32))
        if b == 0:
            vp, up = v, u
            b1_eff = _bias_eff(u, W1_u, b1, corr)
        else:
            vp, up, b1_eff = _node_prep(v, u, bp['node_func'],
                                        bp['state_func'], W1_u, b1, corr)
        vs_g, vd_g = _sc_gather(vp.astype(jnp.bfloat16), src2, dst2)
        (W2, b2), (W3, b3) = bp['conv_edge'][1], bp['conv_edge'][2]
        enew, eout, sume = _edge_update(
            e, vs_g, vd_g, bp['edge_func'] if b > 0 else None,
            W1_vs, W1_vd, W1_e, b1_eff, W2, b2, W3, b3)
        S = _sc_scatter_add(enew, dst2, zeros32)
        v, u = _node_update(v, vp, S, deg2, up, sume, u,
                            bp['conv_node'], bp['conv_state'])
        e = eout

    ns = p['node_s2s']
    q_n = _node_s2s(v, ns['W_ih'].T, ns['W_hh'].T,
                    ns['b_ih'].reshape(1, -1), ns['b_hh'].reshape(1, -1))
    es = p['edge_s2s']
    wihT, whhT = es['W_ih'].T, es['W_hh'].T
    bih, bhh = es['b_ih'].reshape(1, -1), es['b_hh'].reshape(1, -1)
    m1, z1, r1, h1, c1 = _edge_s2s_pass(e, wihT, whhT, bih, bhh, None)
    m2, z2, r2, h2 = _edge_s2s_pass(e, wihT, whhT, bih, bhh,
                                    (m1, z1, r1, h1, c1))
    edge_vec = jnp.concatenate([h2, r2 / z2], axis=1).reshape(64)
    return jnp.hstack([q_n.reshape(64), edge_vec, u.reshape(32)])


# speedup vs baseline: 2.2077x; 2.1834x over previous
import functools

import numpy as np
import jax
import jax.numpy as jnp
from jax import lax
from jax.experimental import pallas as pl
from jax.experimental.pallas import tpu as pltpu
from jax.experimental.pallas import tpu_sc as plsc

_LN2 = float(np.log(2.0))
_LOG2E = float(np.log2(np.e))

E_TOTAL = 320000
PE = E_TOTAL // 4
N_NODES = 10000
_NW = 32
_PER_W = E_TOTAL // _NW
_GRP = 80
_GPW = _PER_W // _GRP
_CGRP = 5
_NCH = _GPW // _CGRP
_CH_E = _GRP * _CGRP
N_PAD = 10240
_NPT = N_PAD // 16
_TP = 1000
_NGRID = PE // _TP


def _sp2(x):
    return jnp.maximum(x, 0.0) + jnp.log(1.0 + jnp.exp(-jnp.abs(x))) - _LN2


def _g2(x):
    return jnp.maximum(x, 0.0) + jnp.log2(1.0 + jnp.exp2(jnp.minimum(x, -x)))


def _sigmoid(x):
    return 1.0 / (1.0 + jnp.exp(-x))


def _full(shape):
    return pl.BlockSpec(shape, lambda i: tuple(0 for _ in shape))


def _bdot(a, b):
    return jnp.dot(a, b, preferred_element_type=jnp.float32)


def _blk4(W):
    return jnp.kron(jnp.eye(4, dtype=W.dtype), W)


def _csum(Wb):
    return jnp.sum(Wb.astype(jnp.float32), axis=0)


def _edge_encoder(edge_feat, W0, b0, W1, b1):
    E, D = edge_feat.shape
    bf = jnp.bfloat16
    W0s = (W0 * _LOG2E).astype(bf)
    W1b = W1.astype(bf)
    W1blk = _blk4(W1b)
    b0s = jnp.tile(b0 * _LOG2E, 4).reshape(1, -1)
    b1s = jnp.tile(b1 * _LOG2E - _csum(W1b), 4).reshape(1, -1)

    def body(x_ref, w0, b0r, w1, b1r, o_ref):
        hs = [_bdot(x_ref[pl.ds(j * _TP, _TP), :].astype(bf), w0[...])
              for j in range(4)]
        a = _g2(jnp.concatenate(hs, axis=1) + b0r[...])
        a = _g2(_bdot(a.astype(bf), w1[...]) + b1r[...])
        o_ref[...] = (a - 1.0) * _LN2

    return pl.pallas_call(
        body,
        grid=(_NGRID,),
        in_specs=[pl.BlockSpec((4 * _TP, D), lambda i: (i, 0)),
                  _full(W0s.shape), _full((1, 256)),
                  _full((256, 128)), _full((1, 128))],
        out_specs=pl.BlockSpec((_TP, 128), lambda i: (i, 0)),
        out_shape=jax.ShapeDtypeStruct((PE, 128), jnp.float32),
    )(edge_feat, W0s, b0s, W1blk, b1s)


def _node_state_encoder(nf2, emb, nW0, nb0, nW1, nb1, st, sW0, sb0, sW1, sb1):
    ntypes = emb.shape[0]

    def body(nf_ref, emb_ref, nw0, nb0r, nw1, nb1r, st_ref, sw0, sb0r, sw1,
             sb1r, v_ref, u_ref):
        ids = nf_ref[...]
        oh = (ids == lax.broadcasted_iota(jnp.int32, (1, ntypes), 1)
              ).astype(jnp.float32)
        v = oh @ emb_ref[...]
        v = _sp2(v @ nw0[...] + nb0r[...])
        v = _sp2(v @ nw1[...] + nb1r[...])
        v_ref[...] = v
        u = _sp2(st_ref[...] @ sw0[...] + sb0r[...])
        u_ref[...] = _sp2(u @ sw1[...] + sb1r[...])

    return pl.pallas_call(
        body,
        out_shape=[jax.ShapeDtypeStruct((N_NODES, 32), jnp.float32),
                   jax.ShapeDtypeStruct((1, 32), jnp.float32)],
    )(nf2, emb, nW0, nb0.reshape(1, -1), nW1, nb1.reshape(1, -1),
      st, sW0, sb0.reshape(1, -1), sW1, sb1.reshape(1, -1))


def _bias_eff(u, W1_u, b1, corr):
    def body(u_ref, wu, b1r, corr_r, be_ref):
        be_ref[...] = ((u_ref[...] @ wu[...] + b1r[...]) * _LOG2E
                       - corr_r[...])

    return pl.pallas_call(
        body, out_shape=jax.ShapeDtypeStruct((1, 64), jnp.float32),
    )(u, W1_u, b1.reshape(1, -1), corr)


def _node_prep(v_in, u_in, node_func, state_func, W1_u, b1, corr):
    (fW0, fb0), (fW1, fb1) = node_func
    (gW0, gb0), (gW1, gb1) = state_func

    def body(v_ref, u_ref, fw0, fb0r, fw1, fb1r, gw0, gb0r, gw1, gb1r, wu,
             b1r, corr_r, vp_ref, up_ref, be_ref):
        vp = _sp2(v_ref[...] @ fw0[...] + fb0r[...])
        vp = _sp2(vp @ fw1[...] + fb1r[...])
        vp_ref[...] = vp
        up = _sp2(u_ref[...] @ gw0[...] + gb0r[...])
        up = _sp2(up @ gw1[...] + gb1r[...])
        up_ref[...] = up
        be_ref[...] = ((up @ wu[...] + b1r[...]) * _LOG2E - corr_r[...])

    return pl.pallas_call(
        body,
        out_shape=[jax.ShapeDtypeStruct((N_NODES, 32), jnp.float32),
                   jax.ShapeDtypeStruct((1, 32), jnp.float32),
                   jax.ShapeDtypeStruct((1, 64), jnp.float32)],
    )(v_in, u_in, fW0, fb0.reshape(1, -1), fW1, fb1.reshape(1, -1),
      gW0, gb0.reshape(1, -1), gW1, gb1.reshape(1, -1), W1_u,
      b1.reshape(1, -1), corr)


def _edge_update(e_in, vs, vd, edge_func, W1_vs, W1_vd, W1_e, b1_eff,
                 W2, b2, W3, b3):
    has_ef = edge_func is not None
    bf = jnp.bfloat16
    w1vs = _blk4((W1_vs * _LOG2E).astype(bf))
    w1vd = _blk4((W1_vd * _LOG2E).astype(bf))
    if has_ef:
        w1e = _blk4(W1_e.astype(bf))
    else:
        w1e = _blk4((W1_e * _LOG2E).astype(bf))
    W2b, W3b = W2.astype(bf), W3.astype(bf)
    bet = jnp.tile(b1_eff, (1, 4))
    b2s = jnp.tile(b2 * _LOG2E - _csum(W2b), 4).reshape(1, -1)
    b3s = jnp.tile(b3 * _LOG2E - _csum(W3b), 4).reshape(1, -1)

    def body(e_ref, vs_ref, vd_ref, *rest):
        if has_ef:
            (fw0, fb0r, fw1, fb1r, w1s_, w1d_, w1e_, be, w2_, b2r, w3_, b3r,
             enew_ref, eout_ref, sume_ref) = rest
        else:
            (w1s_, w1d_, w1e_, be, w2_, b2r, w3_, b3r,
             enew_ref, eout_ref, sume_ref) = rest
        i = pl.program_id(0)
        e_t = e_ref[...]
        if has_ef:
            a = _g2(_bdot(e_t.astype(bf), fw0[...]) + fb0r[...])
            ep = _g2(_bdot(a.astype(bf), fw1[...]) + fb1r[...])
        else:
            ep = e_t
        x1 = (_bdot(vs_ref[...].astype(bf), w1s_[...])
              + _bdot(vd_ref[...].astype(bf), w1d_[...])
              + _bdot(ep.astype(bf), w1e_[...]) + be[...])
        a1 = _g2(x1)
        a2 = _g2(_bdot(a1.astype(bf), w2_[...]) + b2r[...])
        a3 = _g2(_bdot(a2.astype(bf), w3_[...]) + b3r[...])
        enew = (a3 - 1.0) * _LN2
        enew_ref[...] = enew
        eout_ref[...] = enew + e_t

        @pl.when(i == 0)
        def _():
            sume_ref[...] = jnp.zeros_like(sume_ref)

        sume_ref[...] += jnp.sum(enew, axis=0, keepdims=True)

    tile = pl.BlockSpec((_TP, 128), lambda i: (i, 0))
    w_in = ([_full((128, 256)), _full((1, 256)), _full((256, 128)),
             _full((1, 128))] if has_ef else [])
    w_in += [_full((128, 256)), _full((128, 256)), _full((128, 256)),
             _full((1, 256)), _full((256, 256)), _full((1, 256)),
             _full((256, 128)), _full((1, 128))]
    args = [e_in, vs, vd]
    if has_ef:
        (fW0, fb0), (fW1, fb1) = edge_func
        fW1b = fW1.astype(bf)
        args += [_blk4((fW0 * _LOG2E).astype(bf)),
                 jnp.tile(fb0 * _LOG2E, 4).reshape(1, -1),
                 _blk4(fW1b),
                 jnp.tile(fb1 * _LOG2E - _csum(fW1b), 4).reshape(1, -1)]
    args += [w1vs, w1vd, w1e, bet, _blk4(W2b), b2s, _blk4(W3b), b3s]
    return pl.pallas_call(
        body,
        grid=(_NGRID,),
        in_specs=[tile, tile, tile] + w_in,
        out_specs=[tile, tile, _full((1, 128))],
        out_shape=[jax.ShapeDtypeStruct((PE, 128), jnp.float32),
                   jax.ShapeDtypeStruct((PE, 128), jnp.float32),
                   jax.ShapeDtypeStruct((1, 128), jnp.float32)],
    )(*args)


def _node_update(v_in, vp, Sp, deg2, up, sume, fold, u_in, conv_node,
                 conv_state):
    (cW1, cb1), (cW2, cb2), (cW3, cb3) = conv_node
    (sW1, sb1), (sW2, sb2), (sW3, sb3) = conv_state

    def body(vin_ref, vp_ref, s_ref, d_ref, up_ref, sume_ref, fold_r,
             uin_ref,
             cwa, cwb, cwc, cb1r, cw2, cb2r, cw3, cb3r,
             swa, swb, swc, sb1r, sw2, sb2r, sw3, sb3r,
             vout_ref, uout_ref):
        deg = (d_ref[0][0:N_NODES, 0:1] + d_ref[1][0:N_NODES, 0:1])
        ve = ((s_ref[0][0:N_NODES, :] + s_ref[1][0:N_NODES, :])
              / jnp.maximum(deg, 1.0))
        up_v = up_ref[...]
        h = _sp2(vp_ref[...] @ cwa[...] + ve @ cwb[...]
                 + up_v @ cwc[...] + cb1r[...])
        h = _sp2(h @ cw2[...] + cb2r[...])
        vnew = _sp2(h @ cw3[...] + cb3r[...])
        vout_ref[...] = vnew + vin_ref[...]
        mean_v = jnp.mean(vnew, axis=0, keepdims=True)
        mean_e = jnp.dot(sume_ref[...], fold_r[...],
                         preferred_element_type=jnp.float32) * (1.0 / E_TOTAL)
        s = _sp2(up_v @ swa[...] + mean_v @ swb[...] + mean_e @ swc[...]
                 + sb1r[...])
        s = _sp2(s @ sw2[...] + sb2r[...])
        unew = _sp2(s @ sw3[...] + sb3r[...])
        uout_ref[...] = unew + uin_ref[...]

    return pl.pallas_call(
        body,
        out_shape=[jax.ShapeDtypeStruct((N_NODES, 32), jnp.float32),
                   jax.ShapeDtypeStruct((1, 32), jnp.float32)],
    )(v_in, vp, Sp, deg2, up, sume, fold, u_in,
      cW1[0:32], cW1[32:64], cW1[64:96], cb1.reshape(1, -1),
      cW2, cb2.reshape(1, -1), cW3, cb3.reshape(1, -1),
      sW1[0:32], sW1[32:64], sW1[64:96], sb1.reshape(1, -1),
      sW2, sb2.reshape(1, -1), sW3, sb3.reshape(1, -1))


def _lstm_step(q, h, c, wih, whh, bihr, bhhr):
    g = q @ wih[...] + bihr[...] + h @ whh[...] + bhhr[...]
    i_ = _sigmoid(g[:, 0:32])
    f_ = _sigmoid(g[:, 32:64])
    gg = jnp.tanh(g[:, 64:96])
    o_ = _sigmoid(g[:, 96:128])
    c = f_ * c + i_ * gg
    h = o_ * jnp.tanh(c)
    return h, c


def _node_s2s(feat, wihT, whhT, bih, bhh):
    def body(feat_ref, wih, whh, bihr, bhhr, q_ref):
        feat_v = feat_ref[...]
        h = jnp.zeros((1, 32), jnp.float32)
        c = jnp.zeros((1, 32), jnp.float32)
        q = jnp.zeros((1, 64), jnp.float32)
        for _ in range(2):
            h, c = _lstm_step(q, h, c, wih, whh, bihr, bhhr)
            logits = jnp.sum(feat_v * h, axis=1, keepdims=True)
            m = jnp.max(logits, axis=0, keepdims=True)
            a = jnp.exp(logits - m)
            z = jnp.sum(a, axis=0, keepdims=True)
            r = jnp.sum((a / z) * feat_v, axis=0, keepdims=True)
            q = jnp.concatenate([h, r], axis=1)
        q_ref[...] = q

    return pl.pallas_call(
        body, out_shape=jax.ShapeDtypeStruct((1, 64), jnp.float32),
    )(feat, wihT, whhT, bih, bhh)


def _edge_s2s_pass(feat, wihT, whhT, bih, bhh, blk1T, blk1, fold, prev):
    first = prev is None

    def body(feat_ref, wih, whh, bihr, bhhr, b1t, b1_, fold_r, *rest):
        if first:
            m_ref, z_ref, r_ref, h_ref, c_ref = rest
        else:
            m1, z1, r1p, h1, c1, m_ref, z_ref, r_ref, h_ref = rest
        i = pl.program_id(0)

        @pl.when(i == 0)
        def _():
            if first:
                q = jnp.zeros((1, 64), jnp.float32)
                hp = jnp.zeros((1, 32), jnp.float32)
                cp = jnp.zeros((1, 32), jnp.float32)
            else:
                r1 = (jnp.dot(r1p[...], fold_r[...],
                              preferred_element_type=jnp.float32) / z1[...])
                q = jnp.concatenate([h1[...], r1], axis=1)
                hp = h1[...]
                cp = c1[...]
            h, c = _lstm_step(q, hp, cp, wih, whh, bihr, bhhr)
            h_ref[...] = h
            if first:
                c_ref[...] = c
            m_ref[...] = jnp.full((1, 1), -1e30, jnp.float32)
            z_ref[...] = jnp.zeros((1, 1), jnp.float32)
            r_ref[...] = jnp.zeros((1, 128), jnp.float32)

        h = h_ref[...]
        feat_t = feat_ref[...]
        h_tile = jnp.concatenate([h, h, h, h], axis=1)
        logits = jnp.dot(feat_t * h_tile, b1t[...],
                         preferred_element_type=jnp.float32)
        mt = jnp.max(jnp.max(logits, axis=0, keepdims=True), axis=1,
                     keepdims=True)
        m_old = m_ref[...]
        m_new = jnp.maximum(m_old, mt)
        sc = jnp.exp(m_old - m_new)
        a = jnp.exp(logits - m_new)
        z_ref[...] = (z_ref[...] * sc
                      + jnp.sum(jnp.sum(a, axis=0, keepdims=True), axis=1,
                                keepdims=True))
        ab = jnp.dot(a, b1_[...], preferred_element_type=jnp.float32)
        r_ref[...] = (r_ref[...] * sc
                      + jnp.sum(ab * feat_t, axis=0, keepdims=True))
        m_ref[...] = m_new

    tile = pl.BlockSpec((_TP, 128), lambda i: (i, 0))
    small = [_full((1, 1)), _full((1, 1)), _full((1, 128)), _full((1, 32))]
    in_specs = [tile, _full((64, 128)), _full((32, 128)), _full((1, 128)),
                _full((1, 128)), _full((128, 4)), _full((4, 128)),
                _full((128, 32))]
    out_shape = [jax.ShapeDtypeStruct((1, 1), jnp.float32),
                 jax.ShapeDtypeStruct((1, 1), jnp.float32),
                 jax.ShapeDtypeStruct((1, 128), jnp.float32),
                 jax.ShapeDtypeStruct((1, 32), jnp.float32)]
    out_specs = small[:]
    args = [feat, wihT, whhT, bih, bhh, blk1T, blk1, fold]
    if first:
        out_shape.append(jax.ShapeDtypeStruct((1, 32), jnp.float32))
        out_specs.append(_full((1, 32)))
    else:
        in_specs += small + [_full((1, 32))]
        args += list(prev)
    return pl.pallas_call(
        body, grid=(_NGRID,), in_specs=in_specs, out_specs=out_specs,
        out_shape=out_shape,
    )(*args)


_MESH = dict(core_axis_name="c", subcore_axis_name="s")


def _sc_gather(vtab, src2, dst2):
    mesh = plsc.VectorSubcoreMesh(**_MESH)

    @functools.partial(
        pl.kernel, mesh=mesh,
        compiler_params=pltpu.CompilerParams(use_tc_tiling_on_sc=False),
        out_type=[jax.ShapeDtypeStruct((E_TOTAL, 32), jnp.float32),
                  jax.ShapeDtypeStruct((E_TOTAL, 32), jnp.float32)],
        scratch_types=[pltpu.VMEM((_GPW, _GRP), jnp.int32),
                       pltpu.VMEM((_GPW, _GRP), jnp.int32),
                       pltpu.VMEM((_CH_E, 32), jnp.float32),
                       pltpu.VMEM((_CH_E, 32), jnp.float32),
                       pltpu.SemaphoreType.DMA],
    )
    def k(vtab_ref, src_ref, dst_ref, vs_ref, vd_ref, idx_s, idx_d,
          buf_s, buf_d, sem):
        tab = vtab_ref
        vs32 = vs_ref
        vd32 = vd_ref
        wid = lax.axis_index("s") * 2 + lax.axis_index("c")
        pltpu.sync_copy(src_ref.at[wid], idx_s)
        pltpu.sync_copy(dst_ref.at[wid], idx_d)
        ebase = wid * _PER_W

        def chunk(ci, carry):
            cps = []
            for j in range(_CGRP):
                g = ci * _CGRP + j
                cps.append(pltpu.async_copy(
                    tab.at[idx_s.at[g]],
                    buf_s.at[pl.ds(j * _GRP, _GRP)], sem))
                cps.append(pltpu.async_copy(
                    tab.at[idx_d.at[g]],
                    buf_d.at[pl.ds(j * _GRP, _GRP)], sem))
            for cp in cps:
                cp.wait()
            ob = ebase + ci * _CH_E
            pltpu.sync_copy(buf_s, vs32.at[pl.ds(ob, _CH_E)])
            pltpu.sync_copy(buf_d, vd32.at[pl.ds(ob, _CH_E)])
            return carry

        lax.fori_loop(0, _NCH, chunk, 0)

    return k(vtab, src2, dst2)


def _sc_scatter_add(enew, dst2, zeros32):
    mesh = plsc.VectorSubcoreMesh(**_MESH)

    @functools.partial(
        pl.kernel, mesh=mesh,
        compiler_params=pltpu.CompilerParams(use_tc_tiling_on_sc=False),
        out_type=jax.ShapeDtypeStruct((2 * N_PAD, 32), jnp.float32),
        scratch_types=[pltpu.VMEM((_GPW, _GRP), jnp.int32),
                       pltpu.VMEM((_CH_E, 32), jnp.float32),
                       pltpu.VMEM_SHARED((N_PAD, 32), jnp.float32),
                       pltpu.SemaphoreType.DMA],
    )
    def k(enew_ref, dst_ref, z_ref, out_ref, idx_d, rows, accum, sem):
        e32 = enew_ref
        o32 = out_ref
        cid = lax.axis_index("c")
        sid = lax.axis_index("s")
        wid = sid * 2 + cid
        pltpu.sync_copy(dst_ref.at[wid], idx_d)
        pltpu.sync_copy(z_ref.at[pl.ds(sid * _NPT, _NPT)],
                        accum.at[pl.ds(sid * _NPT, _NPT)])
        plsc.subcore_barrier()
        ebase = wid * _PER_W

        def chunk(ci, carry):
            pltpu.sync_copy(e32.at[pl.ds(ebase + ci * _CH_E, _CH_E)], rows)
            cps = [pltpu.async_copy(rows.at[pl.ds(j * _GRP, _GRP)],
                                    accum.at[idx_d.at[ci * _CGRP + j]],
                                    sem, add=True)
                   for j in range(_CGRP)]
            for cp in cps:
                cp.wait()
            return carry

        lax.fori_loop(0, _NCH, chunk, 0)
        plsc.subcore_barrier()
        pltpu.sync_copy(accum.at[pl.ds(sid * _NPT, _NPT)],
                        o32.at[pl.ds(cid * N_PAD + sid * _NPT, _NPT)])

    return k(enew, dst2, zeros32)


def _sc_degree(dst2, ones16, zeros16):
    mesh = plsc.VectorSubcoreMesh(**_MESH)

    @functools.partial(
        pl.kernel, mesh=mesh,
        compiler_params=pltpu.CompilerParams(use_tc_tiling_on_sc=False),
        out_type=jax.ShapeDtypeStruct((2 * N_PAD, 16), jnp.float32),
        scratch_types=[pltpu.VMEM((_GPW, _GRP), jnp.int32),
                       pltpu.VMEM((_GRP, 16), jnp.float32),
                       pltpu.VMEM_SHARED((N_PAD, 16), jnp.float32),
                       pltpu.SemaphoreType.DMA],
    )
    def k(dst_ref, ones_ref, z_ref, out_ref, idx_d, onesbuf, accum, sem):
        cid = lax.axis_index("c")
        sid = lax.axis_index("s")
        wid = sid * 2 + cid
        pltpu.sync_copy(dst_ref.at[wid], idx_d)
        pltpu.sync_copy(ones_ref, onesbuf)
        pltpu.sync_copy(z_ref.at[pl.ds(sid * _NPT, _NPT)],
                        accum.at[pl.ds(sid * _NPT, _NPT)])
        plsc.subcore_barrier()

        def chunk(ci, carry):
            cps = [pltpu.async_copy(onesbuf,
                                    accum.at[idx_d.at[ci * _CGRP + j]],
                                    sem, add=True)
                   for j in range(_CGRP)]
            for cp in cps:
                cp.wait()
            return carry

        lax.fori_loop(0, _NCH, chunk, 0)
        plsc.subcore_barrier()
        pltpu.sync_copy(accum.at[pl.ds(sid * _NPT, _NPT)],
                        out_ref.at[pl.ds(cid * N_PAD + sid * _NPT, _NPT)])

    return k(dst2, ones16, zeros16).reshape(2, N_PAD, 16)


def kernel(edge_index, edge_feat, node_feat, state_feat, params):
    def _perm(x):
        return jnp.transpose(x.reshape(_NGRID, 4, _TP), (0, 2, 1)
                             ).reshape(_NW, _GPW, _GRP)

    src2 = _perm(edge_index[0])
    dst2 = _perm(edge_index[1])
    nf2 = node_feat.reshape(N_NODES, 1)
    fold = jnp.kron(jnp.ones((4, 1), jnp.float32),
                    jnp.eye(32, dtype=jnp.float32))
    zeros32 = jnp.zeros((N_PAD, 32), jnp.float32)
    zeros16 = jnp.zeros((N_PAD, 16), jnp.float32)
    ones16 = jnp.ones((_GRP, 16), jnp.float32)

    p = params
    (eW0, eb0), (eW1, eb1) = p['edge_enc']
    e = _edge_encoder(edge_feat, eW0, eb0, eW1, eb1)
    (nW0, nb0), (nW1, nb1) = p['node_enc']
    (sW0, sb0), (sW1, sb1) = p['state_enc']
    v, u = _node_state_encoder(nf2, p['node_emb'], nW0, nb0, nW1, nb1,
                                     state_feat, sW0, sb0, sW1, sb1)

    deg2 = _sc_degree(dst2, ones16, zeros16)

    for b, bp in enumerate(p['blocks']):
        W1, b1 = bp['conv_edge'][0]
        W1_vs, W1_vd, W1_e, W1_u = W1[0:32], W1[32:64], W1[64:96], W1[96:128]
        corr = (jnp.sum(W1_e.astype(jnp.bfloat16).astype(jnp.float32),
                        axis=0).reshape(1, -1) if b > 0
                else jnp.zeros((1, 64), jnp.float32))
        if b == 0:
            vp, up = v, u
            b1_eff = _bias_eff(u, W1_u, b1, corr)
        else:
            vp, up, b1_eff = _node_prep(v, u, bp['node_func'],
                                        bp['state_func'], W1_u, b1, corr)
        vs_g, vd_g = _sc_gather(vp, src2, dst2)
        (W2, b2), (W3, b3) = bp['conv_edge'][1], bp['conv_edge'][2]
        enew, eout, sume = _edge_update(
            e, vs_g.reshape(PE, 128), vd_g.reshape(PE, 128),
            bp['edge_func'] if b > 0 else None,
            W1_vs, W1_vd, W1_e, b1_eff, W2, b2, W3, b3)
        Sp = _sc_scatter_add(enew.reshape(E_TOTAL, 32), dst2,
                             zeros32).reshape(2, N_PAD, 32)
        v, u = _node_update(v, vp, Sp, deg2, up, sume, fold, u,
                            bp['conv_node'], bp['conv_state'])
        e = eout

    ns = p['node_s2s']
    q_n = _node_s2s(v, ns['W_ih'].T, ns['W_hh'].T,
                    ns['b_ih'].reshape(1, -1), ns['b_hh'].reshape(1, -1))
    es = p['edge_s2s']
    wihT, whhT = es['W_ih'].T, es['W_hh'].T
    bih, bhh = es['b_ih'].reshape(1, -1), es['b_hh'].reshape(1, -1)
    blk1 = jnp.kron(jnp.eye(4, dtype=jnp.float32),
                    jnp.ones((1, 32), jnp.float32))
    blk1T = jnp.kron(jnp.eye(4, dtype=jnp.float32),
                     jnp.ones((32, 1), jnp.float32))
    p1 = _edge_s2s_pass(e, wihT, whhT, bih, bhh, blk1T, blk1, fold, None)
    m2, z2, r2p, h2 = _edge_s2s_pass(e, wihT, whhT, bih, bhh, blk1T, blk1,
                                     fold, p1)
    r2 = r2p.reshape(4, 32).sum(axis=0, keepdims=True) / z2
    edge_vec = jnp.concatenate([h2, r2], axis=1).reshape(64)
    return jnp.hstack([q_n.reshape(64), edge_vec, u.reshape(32)])


# speedup vs baseline: 2.4197x; 1.0960x over previous
import functools

import numpy as np
import jax
import jax.numpy as jnp
from jax import lax
from jax.experimental import pallas as pl
from jax.experimental.pallas import tpu as pltpu
from jax.experimental.pallas import tpu_sc as plsc

_LN2 = float(np.log(2.0))
_LOG2E = float(np.log2(np.e))

E_TOTAL = 320000
PE = E_TOTAL // 4
N_NODES = 10000
_NW = 32
_PER_W = E_TOTAL // _NW
_GRP = 80
_GPW = _PER_W // _GRP
_CGRP = 5
_NCH = _GPW // _CGRP
_CH_E = _GRP * _CGRP
N_PAD = 10240
_NPT = N_PAD // 16
_TP = 1000
_NGRID = PE // _TP
_TP4 = 2000
_TS = 4000


def _sp2(x):
    return jnp.maximum(x, 0.0) + jnp.log(1.0 + jnp.exp(-jnp.abs(x))) - _LN2


def _g2(x):
    return jnp.maximum(x, 0.0) + jnp.log2(1.0 + jnp.exp2(jnp.minimum(x, -x)))


def _sigmoid(x):
    return 1.0 / (1.0 + jnp.exp(-x))


def _full(shape):
    return pl.BlockSpec(shape, lambda i: tuple(0 for _ in shape))


def _bdot(a, b):
    return jnp.dot(a, b, preferred_element_type=jnp.float32)


def _blk4(W):
    return jnp.kron(jnp.eye(4, dtype=W.dtype), W)


def _csum(Wb):
    return jnp.sum(Wb.astype(jnp.float32), axis=0)


def _edge_encoder(edge_feat, W0, b0, W1, b1):
    E, D = edge_feat.shape
    bf = jnp.bfloat16
    W0s = (W0 * _LOG2E).astype(bf)
    W1b = W1.astype(bf)
    W1blk = _blk4(W1b)
    b0s = jnp.tile(b0 * _LOG2E, 4).reshape(1, -1)
    b1s = jnp.tile(b1 * _LOG2E - _csum(W1b), 4).reshape(1, -1)

    def body(x_ref, w0, b0r, w1, b1r, o_ref):
        hs = [_bdot(x_ref[pl.ds(j * _TP, _TP), :].astype(bf), w0[...])
              for j in range(4)]
        a = _g2(jnp.concatenate(hs, axis=1) + b0r[...])
        a = _g2(_bdot(a.astype(bf), w1[...]) + b1r[...])
        o_ref[...] = (a - 1.0) * _LN2

    return pl.pallas_call(
        body,
        grid=(_NGRID,),
        in_specs=[pl.BlockSpec((4 * _TP, D), lambda i: (i, 0)),
                  _full(W0s.shape), _full((1, 256)),
                  _full((256, 128)), _full((1, 128))],
        out_specs=pl.BlockSpec((_TP, 128), lambda i: (i, 0)),
        out_shape=jax.ShapeDtypeStruct((PE, 128), jnp.float32),
    )(edge_feat, W0s, b0s, W1blk, b1s)


def _node_state_encoder(nf2, emb, nW0, nb0, nW1, nb1, st, sW0, sb0, sW1, sb1):
    ntypes = emb.shape[0]

    def body(nf_ref, emb_ref, nw0, nb0r, nw1, nb1r, st_ref, sw0, sb0r, sw1,
             sb1r, v_ref, u_ref):
        ids = nf_ref[...]
        oh = (ids == lax.broadcasted_iota(jnp.int32, (1, ntypes), 1)
              ).astype(jnp.float32)
        v = oh @ emb_ref[...]
        v = _sp2(v @ nw0[...] + nb0r[...])
        v = _sp2(v @ nw1[...] + nb1r[...])
        v_ref[...] = v
        u = _sp2(st_ref[...] @ sw0[...] + sb0r[...])
        u_ref[...] = _sp2(u @ sw1[...] + sb1r[...])

    return pl.pallas_call(
        body,
        out_shape=[jax.ShapeDtypeStruct((N_NODES, 32), jnp.float32),
                   jax.ShapeDtypeStruct((1, 32), jnp.float32)],
    )(nf2, emb, nW0, nb0.reshape(1, -1), nW1, nb1.reshape(1, -1),
      st, sW0, sb0.reshape(1, -1), sW1, sb1.reshape(1, -1))


def _bias_eff(u, W1_u, b1, corr):
    def body(u_ref, wu, b1r, corr_r, be_ref):
        be_ref[...] = ((u_ref[...] @ wu[...] + b1r[...]) * _LOG2E
                       - corr_r[...])

    return pl.pallas_call(
        body, out_shape=jax.ShapeDtypeStruct((1, 64), jnp.float32),
    )(u, W1_u, b1.reshape(1, -1), corr)


def _node_prep(v_in, u_in, node_func, state_func, W1_u, b1, corr):
    (fW0, fb0), (fW1, fb1) = node_func
    (gW0, gb0), (gW1, gb1) = state_func

    def body(v_ref, u_ref, fw0, fb0r, fw1, fb1r, gw0, gb0r, gw1, gb1r, wu,
             b1r, corr_r, vp_ref, up_ref, be_ref):
        vp = _sp2(v_ref[...] @ fw0[...] + fb0r[...])
        vp = _sp2(vp @ fw1[...] + fb1r[...])
        vp_ref[...] = vp
        up = _sp2(u_ref[...] @ gw0[...] + gb0r[...])
        up = _sp2(up @ gw1[...] + gb1r[...])
        up_ref[...] = up
        be_ref[...] = ((up @ wu[...] + b1r[...]) * _LOG2E - corr_r[...])

    return pl.pallas_call(
        body,
        out_shape=[jax.ShapeDtypeStruct((N_NODES, 32), jnp.float32),
                   jax.ShapeDtypeStruct((1, 32), jnp.float32),
                   jax.ShapeDtypeStruct((1, 64), jnp.float32)],
    )(v_in, u_in, fW0, fb0.reshape(1, -1), fW1, fb1.reshape(1, -1),
      gW0, gb0.reshape(1, -1), gW1, gb1.reshape(1, -1), W1_u,
      b1.reshape(1, -1), corr)


def _edge_update(e_in, vs, vd, edge_func, W1_vs, W1_vd, W1_e, b1_eff,
                 W2, b2, W3, b3):
    has_ef = edge_func is not None
    bf = jnp.bfloat16
    w1vs = _blk4((W1_vs * _LOG2E).astype(bf))
    w1vd = _blk4((W1_vd * _LOG2E).astype(bf))
    if has_ef:
        w1e = _blk4(W1_e.astype(bf))
    else:
        w1e = _blk4((W1_e * _LOG2E).astype(bf))
    W2b, W3b = W2.astype(bf), W3.astype(bf)
    bet = jnp.tile(b1_eff, (1, 4))
    b2s = jnp.tile(b2 * _LOG2E - _csum(W2b), 4).reshape(1, -1)
    b3s = jnp.tile(b3 * _LOG2E - _csum(W3b), 4).reshape(1, -1)

    def body(e_ref, vs_ref, vd_ref, *rest):
        if has_ef:
            (fw0, fb0r, fw1, fb1r, w1s_, w1d_, w1e_, be, w2_, b2r, w3_, b3r,
             enew_ref, eout_ref, sume_ref) = rest
        else:
            (w1s_, w1d_, w1e_, be, w2_, b2r, w3_, b3r,
             enew_ref, eout_ref, sume_ref) = rest
        i = pl.program_id(0)
        e_t = e_ref[...]
        if has_ef:
            a = _g2(_bdot(e_t.astype(bf), fw0[...]) + fb0r[...])
            ep = _g2(_bdot(a.astype(bf), fw1[...]) + fb1r[...])
        else:
            ep = e_t
        x1 = (_bdot(vs_ref[...].astype(bf), w1s_[...])
              + _bdot(vd_ref[...].astype(bf), w1d_[...])
              + _bdot(ep.astype(bf), w1e_[...]) + be[...])
        a1 = _g2(x1)
        a2 = _g2(_bdot(a1.astype(bf), w2_[...]) + b2r[...])
        a3 = _g2(_bdot(a2.astype(bf), w3_[...]) + b3r[...])
        enew = (a3 - 1.0) * _LN2
        enew_ref[...] = enew
        eout_ref[...] = enew + e_t

        @pl.when(i == 0)
        def _():
            sume_ref[...] = jnp.zeros_like(sume_ref)

        sume_ref[...] += jnp.sum(enew, axis=0, keepdims=True)

    tile = pl.BlockSpec((_TP4, 128), lambda i: (i, 0))
    w_in = ([_full((128, 256)), _full((1, 256)), _full((256, 128)),
             _full((1, 128))] if has_ef else [])
    w_in += [_full((128, 256)), _full((128, 256)), _full((128, 256)),
             _full((1, 256)), _full((256, 256)), _full((1, 256)),
             _full((256, 128)), _full((1, 128))]
    args = [e_in, vs, vd]
    if has_ef:
        (fW0, fb0), (fW1, fb1) = edge_func
        fW1b = fW1.astype(bf)
        args += [_blk4((fW0 * _LOG2E).astype(bf)),
                 jnp.tile(fb0 * _LOG2E, 4).reshape(1, -1),
                 _blk4(fW1b),
                 jnp.tile(fb1 * _LOG2E - _csum(fW1b), 4).reshape(1, -1)]
    args += [w1vs, w1vd, w1e, bet, _blk4(W2b), b2s, _blk4(W3b), b3s]
    return pl.pallas_call(
        body,
        grid=(PE // _TP4,),
        in_specs=[tile, tile, tile] + w_in,
        out_specs=[tile, tile, _full((1, 128))],
        out_shape=[jax.ShapeDtypeStruct((PE, 128), jnp.float32),
                   jax.ShapeDtypeStruct((PE, 128), jnp.float32),
                   jax.ShapeDtypeStruct((1, 128), jnp.float32)],
    )(*args)


def _node_update(v_in, vp, Sp, deg2, up, sume, fold, u_in, conv_node,
                 conv_state):
    (cW1, cb1), (cW2, cb2), (cW3, cb3) = conv_node
    (sW1, sb1), (sW2, sb2), (sW3, sb3) = conv_state

    def body(vin_ref, vp_ref, s_ref, d_ref, up_ref, sume_ref, fold_r,
             uin_ref,
             cwa, cwb, cwc, cb1r, cw2, cb2r, cw3, cb3r,
             swa, swb, swc, sb1r, sw2, sb2r, sw3, sb3r,
             vout_ref, uout_ref):
        deg = (d_ref[0][0:N_NODES, 0:1] + d_ref[1][0:N_NODES, 0:1])
        ve = ((s_ref[0][0:N_NODES, :] + s_ref[1][0:N_NODES, :])
              / jnp.maximum(deg, 1.0))
        up_v = up_ref[...]
        h = _sp2(vp_ref[...] @ cwa[...] + ve @ cwb[...]
                 + up_v @ cwc[...] + cb1r[...])
        h = _sp2(h @ cw2[...] + cb2r[...])
        vnew = _sp2(h @ cw3[...] + cb3r[...])
        vout_ref[...] = vnew + vin_ref[...]
        mean_v = jnp.mean(vnew, axis=0, keepdims=True)
        mean_e = jnp.dot(sume_ref[...], fold_r[...],
                         preferred_element_type=jnp.float32) * (1.0 / E_TOTAL)
        s = _sp2(up_v @ swa[...] + mean_v @ swb[...] + mean_e @ swc[...]
                 + sb1r[...])
        s = _sp2(s @ sw2[...] + sb2r[...])
        unew = _sp2(s @ sw3[...] + sb3r[...])
        uout_ref[...] = unew + uin_ref[...]

    return pl.pallas_call(
        body,
        out_shape=[jax.ShapeDtypeStruct((N_NODES, 32), jnp.float32),
                   jax.ShapeDtypeStruct((1, 32), jnp.float32)],
    )(v_in, vp, Sp, deg2, up, sume, fold, u_in,
      cW1[0:32], cW1[32:64], cW1[64:96], cb1.reshape(1, -1),
      cW2, cb2.reshape(1, -1), cW3, cb3.reshape(1, -1),
      sW1[0:32], sW1[32:64], sW1[64:96], sb1.reshape(1, -1),
      sW2, sb2.reshape(1, -1), sW3, sb3.reshape(1, -1))


def _lstm_step(q, h, c, wih, whh, bihr, bhhr):
    g = q @ wih[...] + bihr[...] + h @ whh[...] + bhhr[...]
    i_ = _sigmoid(g[:, 0:32])
    f_ = _sigmoid(g[:, 32:64])
    gg = jnp.tanh(g[:, 64:96])
    o_ = _sigmoid(g[:, 96:128])
    c = f_ * c + i_ * gg
    h = o_ * jnp.tanh(c)
    return h, c


def _node_s2s(feat, wihT, whhT, bih, bhh):
    def body(feat_ref, wih, whh, bihr, bhhr, q_ref):
        feat_v = feat_ref[...]
        h = jnp.zeros((1, 32), jnp.float32)
        c = jnp.zeros((1, 32), jnp.float32)
        q = jnp.zeros((1, 64), jnp.float32)
        for _ in range(2):
            h, c = _lstm_step(q, h, c, wih, whh, bihr, bhhr)
            logits = jnp.sum(feat_v * h, axis=1, keepdims=True)
            m = jnp.max(logits, axis=0, keepdims=True)
            a = jnp.exp(logits - m)
            z = jnp.sum(a, axis=0, keepdims=True)
            r = jnp.sum((a / z) * feat_v, axis=0, keepdims=True)
            q = jnp.concatenate([h, r], axis=1)
        q_ref[...] = q

    return pl.pallas_call(
        body, out_shape=jax.ShapeDtypeStruct((1, 64), jnp.float32),
    )(feat, wihT, whhT, bih, bhh)


def _edge_s2s_pass(feat, wihT, whhT, bih, bhh, blk1T, blk1, fold, prev):
    first = prev is None

    def body(feat_ref, wih, whh, bihr, bhhr, b1t, b1_, fold_r, *rest):
        if first:
            m_ref, z_ref, r_ref, h_ref, c_ref = rest
        else:
            m1, z1, r1p, h1, c1, m_ref, z_ref, r_ref, h_ref = rest
        i = pl.program_id(0)

        @pl.when(i == 0)
        def _():
            if first:
                q = jnp.zeros((1, 64), jnp.float32)
                hp = jnp.zeros((1, 32), jnp.float32)
                cp = jnp.zeros((1, 32), jnp.float32)
            else:
                r1 = (jnp.dot(r1p[...], fold_r[...],
                              preferred_element_type=jnp.float32) / z1[...])
                q = jnp.concatenate([h1[...], r1], axis=1)
                hp = h1[...]
                cp = c1[...]
            h, c = _lstm_step(q, hp, cp, wih, whh, bihr, bhhr)
            h_ref[...] = h
            if first:
                c_ref[...] = c
            m_ref[...] = jnp.full((1, 1), -1e30, jnp.float32)
            z_ref[...] = jnp.zeros((1, 1), jnp.float32)
            r_ref[...] = jnp.zeros((1, 128), jnp.float32)

        h = h_ref[...]
        feat_t = feat_ref[...]
        h_tile = jnp.concatenate([h, h, h, h], axis=1)
        logits = jnp.dot(feat_t * h_tile, b1t[...],
                         preferred_element_type=jnp.float32)
        mt = jnp.max(jnp.max(logits, axis=0, keepdims=True), axis=1,
                     keepdims=True)
        m_old = m_ref[...]
        m_new = jnp.maximum(m_old, mt)
        sc = jnp.exp(m_old - m_new)
        a = jnp.exp(logits - m_new)
        z_ref[...] = (z_ref[...] * sc
                      + jnp.sum(jnp.sum(a, axis=0, keepdims=True), axis=1,
                                keepdims=True))
        ab = jnp.dot(a, b1_[...], preferred_element_type=jnp.float32)
        r_ref[...] = (r_ref[...] * sc
                      + jnp.sum(ab * feat_t, axis=0, keepdims=True))
        m_ref[...] = m_new

    tile = pl.BlockSpec((_TS, 128), lambda i: (i, 0))
    small = [_full((1, 1)), _full((1, 1)), _full((1, 128)), _full((1, 32))]
    in_specs = [tile, _full((64, 128)), _full((32, 128)), _full((1, 128)),
                _full((1, 128)), _full((128, 4)), _full((4, 128)),
                _full((128, 32))]
    out_shape = [jax.ShapeDtypeStruct((1, 1), jnp.float32),
                 jax.ShapeDtypeStruct((1, 1), jnp.float32),
                 jax.ShapeDtypeStruct((1, 128), jnp.float32),
                 jax.ShapeDtypeStruct((1, 32), jnp.float32)]
    out_specs = small[:]
    args = [feat, wihT, whhT, bih, bhh, blk1T, blk1, fold]
    if first:
        out_shape.append(jax.ShapeDtypeStruct((1, 32), jnp.float32))
        out_specs.append(_full((1, 32)))
    else:
        in_specs += small + [_full((1, 32))]
        args += list(prev)
    return pl.pallas_call(
        body, grid=(PE // _TS,), in_specs=in_specs, out_specs=out_specs,
        out_shape=out_shape,
    )(*args)


_MESH = dict(core_axis_name="c", subcore_axis_name="s")


def _sc_gather(vtab, src2, dst2):
    mesh = plsc.VectorSubcoreMesh(**_MESH)

    @functools.partial(
        pl.kernel, mesh=mesh,
        compiler_params=pltpu.CompilerParams(use_tc_tiling_on_sc=False),
        out_type=[jax.ShapeDtypeStruct((E_TOTAL, 32), jnp.float32),
                  jax.ShapeDtypeStruct((E_TOTAL, 32), jnp.float32)],
        scratch_types=[pltpu.VMEM((_GPW, _GRP), jnp.int32),
                       pltpu.VMEM((_GPW, _GRP), jnp.int32),
                       pltpu.VMEM((_CH_E, 32), jnp.float32),
                       pltpu.VMEM((_CH_E, 32), jnp.float32),
                       pltpu.SemaphoreType.DMA],
    )
    def k(vtab_ref, src_ref, dst_ref, vs_ref, vd_ref, idx_s, idx_d,
          buf_s, buf_d, sem):
        tab = vtab_ref
        vs32 = vs_ref
        vd32 = vd_ref
        wid = lax.axis_index("s") * 2 + lax.axis_index("c")
        pltpu.sync_copy(src_ref.at[wid], idx_s)
        pltpu.sync_copy(dst_ref.at[wid], idx_d)
        ebase = wid * _PER_W

        def chunk(ci, carry):
            cps = []
            for j in range(_CGRP):
                g = ci * _CGRP + j
                cps.append(pltpu.async_copy(
                    tab.at[idx_s.at[g]],
                    buf_s.at[pl.ds(j * _GRP, _GRP)], sem))
                cps.append(pltpu.async_copy(
                    tab.at[idx_d.at[g]],
                    buf_d.at[pl.ds(j * _GRP, _GRP)], sem))
            for cp in cps:
                cp.wait()
            ob = ebase + ci * _CH_E
            pltpu.sync_copy(buf_s, vs32.at[pl.ds(ob, _CH_E)])
            pltpu.sync_copy(buf_d, vd32.at[pl.ds(ob, _CH_E)])
            return carry

        lax.fori_loop(0, _NCH, chunk, 0)

    return k(vtab, src2, dst2)


def _sc_scatter_add(enew, dst2, zeros32):
    mesh = plsc.VectorSubcoreMesh(**_MESH)

    @functools.partial(
        pl.kernel, mesh=mesh,
        compiler_params=pltpu.CompilerParams(use_tc_tiling_on_sc=False),
        out_type=jax.ShapeDtypeStruct((2 * N_PAD, 32), jnp.float32),
        scratch_types=[pltpu.VMEM((_GPW, _GRP), jnp.int32),
                       pltpu.VMEM((_CH_E, 32), jnp.float32),
                       pltpu.VMEM_SHARED((N_PAD, 32), jnp.float32),
                       pltpu.SemaphoreType.DMA],
    )
    def k(enew_ref, dst_ref, z_ref, out_ref, idx_d, rows, accum, sem):
        e32 = enew_ref
        o32 = out_ref
        cid = lax.axis_index("c")
        sid = lax.axis_index("s")
        wid = sid * 2 + cid
        pltpu.sync_copy(dst_ref.at[wid], idx_d)
        pltpu.sync_copy(z_ref.at[pl.ds(sid * _NPT, _NPT)],
                        accum.at[pl.ds(sid * _NPT, _NPT)])
        plsc.subcore_barrier()
        ebase = wid * _PER_W

        def chunk(ci, carry):
            pltpu.sync_copy(e32.at[pl.ds(ebase + ci * _CH_E, _CH_E)], rows)
            cps = [pltpu.async_copy(rows.at[pl.ds(j * _GRP, _GRP)],
                                    accum.at[idx_d.at[ci * _CGRP + j]],
                                    sem, add=True)
                   for j in range(_CGRP)]
            for cp in cps:
                cp.wait()
            return carry

        lax.fori_loop(0, _NCH, chunk, 0)
        plsc.subcore_barrier()
        pltpu.sync_copy(accum.at[pl.ds(sid * _NPT, _NPT)],
                        o32.at[pl.ds(cid * N_PAD + sid * _NPT, _NPT)])

    return k(enew, dst2, zeros32)


def _sc_degree(dst2, ones16, zeros16):
    mesh = plsc.VectorSubcoreMesh(**_MESH)

    @functools.partial(
        pl.kernel, mesh=mesh,
        compiler_params=pltpu.CompilerParams(use_tc_tiling_on_sc=False),
        out_type=jax.ShapeDtypeStruct((2 * N_PAD, 16), jnp.float32),
        scratch_types=[pltpu.VMEM((_GPW, _GRP), jnp.int32),
                       pltpu.VMEM((_GRP, 16), jnp.float32),
                       pltpu.VMEM_SHARED((N_PAD, 16), jnp.float32),
                       pltpu.SemaphoreType.DMA],
    )
    def k(dst_ref, ones_ref, z_ref, out_ref, idx_d, onesbuf, accum, sem):
        cid = lax.axis_index("c")
        sid = lax.axis_index("s")
        wid = sid * 2 + cid
        pltpu.sync_copy(dst_ref.at[wid], idx_d)
        pltpu.sync_copy(ones_ref, onesbuf)
        pltpu.sync_copy(z_ref.at[pl.ds(sid * _NPT, _NPT)],
                        accum.at[pl.ds(sid * _NPT, _NPT)])
        plsc.subcore_barrier()

        def chunk(ci, carry):
            cps = [pltpu.async_copy(onesbuf,
                                    accum.at[idx_d.at[ci * _CGRP + j]],
                                    sem, add=True)
                   for j in range(_CGRP)]
            for cp in cps:
                cp.wait()
            return carry

        lax.fori_loop(0, _NCH, chunk, 0)
        plsc.subcore_barrier()
        pltpu.sync_copy(accum.at[pl.ds(sid * _NPT, _NPT)],
                        out_ref.at[pl.ds(cid * N_PAD + sid * _NPT, _NPT)])

    return k(dst2, ones16, zeros16).reshape(2, N_PAD, 16)


def kernel(edge_index, edge_feat, node_feat, state_feat, params):
    def _perm(x):
        return jnp.transpose(x.reshape(_NGRID, 4, _TP), (0, 2, 1)
                             ).reshape(_NW, _GPW, _GRP)

    src2 = _perm(edge_index[0])
    dst2 = _perm(edge_index[1])
    nf2 = node_feat.reshape(N_NODES, 1)
    fold = jnp.kron(jnp.ones((4, 1), jnp.float32),
                    jnp.eye(32, dtype=jnp.float32))
    zeros32 = jnp.zeros((N_PAD, 32), jnp.float32)
    zeros16 = jnp.zeros((N_PAD, 16), jnp.float32)
    ones16 = jnp.ones((_GRP, 16), jnp.float32)

    p = params
    (eW0, eb0), (eW1, eb1) = p['edge_enc']
    e = _edge_encoder(edge_feat, eW0, eb0, eW1, eb1)
    (nW0, nb0), (nW1, nb1) = p['node_enc']
    (sW0, sb0), (sW1, sb1) = p['state_enc']
    v, u = _node_state_encoder(nf2, p['node_emb'], nW0, nb0, nW1, nb1,
                                     state_feat, sW0, sb0, sW1, sb1)

    deg2 = _sc_degree(dst2, ones16, zeros16)

    for b, bp in enumerate(p['blocks']):
        W1, b1 = bp['conv_edge'][0]
        W1_vs, W1_vd, W1_e, W1_u = W1[0:32], W1[32:64], W1[64:96], W1[96:128]
        corr = (jnp.sum(W1_e.astype(jnp.bfloat16).astype(jnp.float32),
                        axis=0).reshape(1, -1) if b > 0
                else jnp.zeros((1, 64), jnp.float32))
        if b == 0:
            vp, up = v, u
            b1_eff = _bias_eff(u, W1_u, b1, corr)
        else:
            vp, up, b1_eff = _node_prep(v, u, bp['node_func'],
                                        bp['state_func'], W1_u, b1, corr)
        vs_g, vd_g = _sc_gather(vp, src2, dst2)
        (W2, b2), (W3, b3) = bp['conv_edge'][1], bp['conv_edge'][2]
        enew, eout, sume = _edge_update(
            e, vs_g.reshape(PE, 128), vd_g.reshape(PE, 128),
            bp['edge_func'] if b > 0 else None,
            W1_vs, W1_vd, W1_e, b1_eff, W2, b2, W3, b3)
        Sp = _sc_scatter_add(enew.reshape(E_TOTAL, 32), dst2,
                             zeros32).reshape(2, N_PAD, 32)
        v, u = _node_update(v, vp, Sp, deg2, up, sume, fold, u,
                            bp['conv_node'], bp['conv_state'])
        e = eout

    ns = p['node_s2s']
    q_n = _node_s2s(v, ns['W_ih'].T, ns['W_hh'].T,
                    ns['b_ih'].reshape(1, -1), ns['b_hh'].reshape(1, -1))
    es = p['edge_s2s']
    wihT, whhT = es['W_ih'].T, es['W_hh'].T
    bih, bhh = es['b_ih'].reshape(1, -1), es['b_hh'].reshape(1, -1)
    blk1 = jnp.kron(jnp.eye(4, dtype=jnp.float32),
                    jnp.ones((1, 32), jnp.float32))
    blk1T = jnp.kron(jnp.eye(4, dtype=jnp.float32),
                     jnp.ones((32, 1), jnp.float32))
    p1 = _edge_s2s_pass(e, wihT, whhT, bih, bhh, blk1T, blk1, fold, None)
    m2, z2, r2p, h2 = _edge_s2s_pass(e, wihT, whhT, bih, bhh, blk1T, blk1,
                                     fold, p1)
    r2 = r2p.reshape(4, 32).sum(axis=0, keepdims=True) / z2
    edge_vec = jnp.concatenate([h2, r2], axis=1).reshape(64)
    return jnp.hstack([q_n.reshape(64), edge_vec, u.reshape(32)])


# speedup vs baseline: 2.4386x; 1.0078x over previous
import functools

import numpy as np
import jax
import jax.numpy as jnp
from jax import lax
from jax.experimental import pallas as pl
from jax.experimental.pallas import tpu as pltpu
from jax.experimental.pallas import tpu_sc as plsc

_LN2 = float(np.log(2.0))
_LOG2E = float(np.log2(np.e))

E_TOTAL = 320000
PE = E_TOTAL // 4
N_NODES = 10000
_NW = 32
_PER_W = E_TOTAL // _NW
_GRP = 80
_GPW = _PER_W // _GRP
_CGRP = 5
_NCH = _GPW // _CGRP
_CH_E = _GRP * _CGRP
N_PAD = 10240
_NPT = N_PAD // 16
_TP = 1000
_NGRID = PE // _TP
_TP4 = 2000
_TS = 4000


def _sp2(x):
    return jnp.maximum(x, 0.0) + jnp.log(1.0 + jnp.exp(-jnp.abs(x))) - _LN2


def _g2(x):
    return jnp.maximum(x, 0.0) + jnp.log2(1.0 + jnp.exp2(jnp.minimum(x, -x)))


def _sigmoid(x):
    return 1.0 / (1.0 + jnp.exp(-x))


def _full(shape):
    return pl.BlockSpec(shape, lambda i: tuple(0 for _ in shape))


def _bdot(a, b):
    return jnp.dot(a, b, preferred_element_type=jnp.float32)


def _blk4(W):
    return jnp.kron(jnp.eye(4, dtype=W.dtype), W)


def _csum(Wb):
    return jnp.sum(Wb.astype(jnp.float32), axis=0)


def _edge_encoder(edge_feat, W0, b0, W1, b1):
    E, D = edge_feat.shape
    W0s = W0 * _LOG2E
    W1blk = _blk4(W1)
    b0s = jnp.tile(b0 * _LOG2E, 4).reshape(1, -1)
    b1s = jnp.tile(b1 * _LOG2E - jnp.sum(W1, axis=0), 4).reshape(1, -1)

    def body(x_ref, w0, b0r, w1, b1r, o_ref):
        hs = [_bdot(x_ref[pl.ds(j * _TP, _TP), :], w0[...])
              for j in range(4)]
        a = _g2(jnp.concatenate(hs, axis=1) + b0r[...])
        a = _g2(_bdot(a, w1[...]) + b1r[...])
        o_ref[...] = (a - 1.0) * _LN2

    return pl.pallas_call(
        body,
        grid=(_NGRID,),
        in_specs=[pl.BlockSpec((4 * _TP, D), lambda i: (i, 0)),
                  _full(W0s.shape), _full((1, 256)),
                  _full((256, 128)), _full((1, 128))],
        out_specs=pl.BlockSpec((_TP, 128), lambda i: (i, 0)),
        out_shape=jax.ShapeDtypeStruct((PE, 128), jnp.float32),
    )(edge_feat, W0s, b0s, W1blk, b1s)


def _node_state_encoder(nf2, emb, nW0, nb0, nW1, nb1, st, sW0, sb0, sW1, sb1):
    ntypes = emb.shape[0]

    def body(nf_ref, emb_ref, nw0, nb0r, nw1, nb1r, st_ref, sw0, sb0r, sw1,
             sb1r, v_ref, u_ref):
        ids = nf_ref[...]
        oh = (ids == lax.broadcasted_iota(jnp.int32, (1, ntypes), 1)
              ).astype(jnp.float32)
        v = oh @ emb_ref[...]
        v = _sp2(v @ nw0[...] + nb0r[...])
        v = _sp2(v @ nw1[...] + nb1r[...])
        v_ref[...] = v
        u = _sp2(st_ref[...] @ sw0[...] + sb0r[...])
        u_ref[...] = _sp2(u @ sw1[...] + sb1r[...])

    return pl.pallas_call(
        body,
        out_shape=[jax.ShapeDtypeStruct((N_NODES, 32), jnp.float32),
                   jax.ShapeDtypeStruct((1, 32), jnp.float32)],
    )(nf2, emb, nW0, nb0.reshape(1, -1), nW1, nb1.reshape(1, -1),
      st, sW0, sb0.reshape(1, -1), sW1, sb1.reshape(1, -1))


def _bias_eff(u, W1_u, b1, corr):
    def body(u_ref, wu, b1r, corr_r, be_ref):
        be_ref[...] = ((u_ref[...] @ wu[...] + b1r[...]) * _LOG2E
                       - corr_r[...])

    return pl.pallas_call(
        body, out_shape=jax.ShapeDtypeStruct((1, 64), jnp.float32),
    )(u, W1_u, b1.reshape(1, -1), corr)


def _node_prep(v_in, u_in, node_func, state_func, W1_u, b1, corr):
    (fW0, fb0), (fW1, fb1) = node_func
    (gW0, gb0), (gW1, gb1) = state_func

    def body(v_ref, u_ref, fw0, fb0r, fw1, fb1r, gw0, gb0r, gw1, gb1r, wu,
             b1r, corr_r, vp_ref, up_ref, be_ref):
        vp = _sp2(v_ref[...] @ fw0[...] + fb0r[...])
        vp = _sp2(vp @ fw1[...] + fb1r[...])
        vp_ref[...] = vp
        up = _sp2(u_ref[...] @ gw0[...] + gb0r[...])
        up = _sp2(up @ gw1[...] + gb1r[...])
        up_ref[...] = up
        be_ref[...] = ((up @ wu[...] + b1r[...]) * _LOG2E - corr_r[...])

    return pl.pallas_call(
        body,
        out_shape=[jax.ShapeDtypeStruct((N_NODES, 32), jnp.float32),
                   jax.ShapeDtypeStruct((1, 32), jnp.float32),
                   jax.ShapeDtypeStruct((1, 64), jnp.float32)],
    )(v_in, u_in, fW0, fb0.reshape(1, -1), fW1, fb1.reshape(1, -1),
      gW0, gb0.reshape(1, -1), gW1, gb1.reshape(1, -1), W1_u,
      b1.reshape(1, -1), corr)


def _edge_update(e_in, vs, vd, edge_func, W1_vs, W1_vd, W1_e, b1_eff,
                 W2, b2, W3, b3):
    has_ef = edge_func is not None
    w1vs = _blk4(W1_vs * _LOG2E)
    w1vd = _blk4(W1_vd * _LOG2E)
    w1e = _blk4(W1_e if has_ef else W1_e * _LOG2E)
    bet = jnp.tile(b1_eff, (1, 4))
    b2s = jnp.tile(b2 * _LOG2E - jnp.sum(W2, axis=0), 4).reshape(1, -1)
    b3s = jnp.tile(b3 * _LOG2E - jnp.sum(W3, axis=0), 4).reshape(1, -1)

    def body(e_ref, vs_ref, vd_ref, *rest):
        if has_ef:
            (fw0, fb0r, fw1, fb1r, w1s_, w1d_, w1e_, be, w2_, b2r, w3_, b3r,
             enew_ref, eout_ref, sume_ref) = rest
        else:
            (w1s_, w1d_, w1e_, be, w2_, b2r, w3_, b3r,
             enew_ref, eout_ref, sume_ref) = rest
        i = pl.program_id(0)
        e_t = e_ref[...]
        if has_ef:
            a = _g2(_bdot(e_t, fw0[...]) + fb0r[...])
            ep = _g2(_bdot(a, fw1[...]) + fb1r[...])
        else:
            ep = e_t
        x1 = (_bdot(vs_ref[...], w1s_[...])
              + _bdot(vd_ref[...], w1d_[...])
              + _bdot(ep, w1e_[...]) + be[...])
        a1 = _g2(x1)
        a2 = _g2(_bdot(a1, w2_[...]) + b2r[...])
        a3 = _g2(_bdot(a2, w3_[...]) + b3r[...])
        enew = (a3 - 1.0) * _LN2
        enew_ref[...] = enew
        eout_ref[...] = enew + e_t

        @pl.when(i == 0)
        def _():
            sume_ref[...] = jnp.zeros_like(sume_ref)

        sume_ref[...] += jnp.sum(enew, axis=0, keepdims=True)

    tile = pl.BlockSpec((_TP4, 128), lambda i: (i, 0))
    w_in = ([_full((128, 256)), _full((1, 256)), _full((256, 128)),
             _full((1, 128))] if has_ef else [])
    w_in += [_full((128, 256)), _full((128, 256)), _full((128, 256)),
             _full((1, 256)), _full((256, 256)), _full((1, 256)),
             _full((256, 128)), _full((1, 128))]
    args = [e_in, vs, vd]
    if has_ef:
        (fW0, fb0), (fW1, fb1) = edge_func
        args += [_blk4(fW0 * _LOG2E),
                 jnp.tile(fb0 * _LOG2E, 4).reshape(1, -1),
                 _blk4(fW1),
                 jnp.tile(fb1 * _LOG2E - jnp.sum(fW1, axis=0),
                          4).reshape(1, -1)]
    args += [w1vs, w1vd, w1e, bet, _blk4(W2), b2s, _blk4(W3), b3s]
    return pl.pallas_call(
        body,
        grid=(PE // _TP4,),
        in_specs=[tile, tile, tile] + w_in,
        out_specs=[tile, tile, _full((1, 128))],
        out_shape=[jax.ShapeDtypeStruct((PE, 128), jnp.float32),
                   jax.ShapeDtypeStruct((PE, 128), jnp.float32),
                   jax.ShapeDtypeStruct((1, 128), jnp.float32)],
    )(*args)


def _node_update(v_in, vp, Sp, deg2, up, sume, fold, u_in, conv_node,
                 conv_state):
    (cW1, cb1), (cW2, cb2), (cW3, cb3) = conv_node
    (sW1, sb1), (sW2, sb2), (sW3, sb3) = conv_state

    def body(vin_ref, vp_ref, s_ref, d_ref, up_ref, sume_ref, fold_r,
             uin_ref,
             cwa, cwb, cwc, cb1r, cw2, cb2r, cw3, cb3r,
             swa, swb, swc, sb1r, sw2, sb2r, sw3, sb3r,
             vout_ref, uout_ref):
        deg = (d_ref[0][0:N_NODES, 0:1] + d_ref[1][0:N_NODES, 0:1])
        ve = ((s_ref[0][0:N_NODES, :] + s_ref[1][0:N_NODES, :])
              / jnp.maximum(deg, 1.0))
        up_v = up_ref[...]
        h = _sp2(vp_ref[...] @ cwa[...] + ve @ cwb[...]
                 + up_v @ cwc[...] + cb1r[...])
        h = _sp2(h @ cw2[...] + cb2r[...])
        vnew = _sp2(h @ cw3[...] + cb3r[...])
        vout_ref[...] = vnew + vin_ref[...]
        mean_v = jnp.mean(vnew, axis=0, keepdims=True)
        mean_e = jnp.dot(sume_ref[...], fold_r[...],
                         preferred_element_type=jnp.float32) * (1.0 / E_TOTAL)
        s = _sp2(up_v @ swa[...] + mean_v @ swb[...] + mean_e @ swc[...]
                 + sb1r[...])
        s = _sp2(s @ sw2[...] + sb2r[...])
        unew = _sp2(s @ sw3[...] + sb3r[...])
        uout_ref[...] = unew + uin_ref[...]

    return pl.pallas_call(
        body,
        out_shape=[jax.ShapeDtypeStruct((N_NODES, 32), jnp.float32),
                   jax.ShapeDtypeStruct((1, 32), jnp.float32)],
    )(v_in, vp, Sp, deg2, up, sume, fold, u_in,
      cW1[0:32], cW1[32:64], cW1[64:96], cb1.reshape(1, -1),
      cW2, cb2.reshape(1, -1), cW3, cb3.reshape(1, -1),
      sW1[0:32], sW1[32:64], sW1[64:96], sb1.reshape(1, -1),
      sW2, sb2.reshape(1, -1), sW3, sb3.reshape(1, -1))


def _lstm_step(q, h, c, wih, whh, bihr, bhhr):
    g = q @ wih[...] + bihr[...] + h @ whh[...] + bhhr[...]
    i_ = _sigmoid(g[:, 0:32])
    f_ = _sigmoid(g[:, 32:64])
    gg = jnp.tanh(g[:, 64:96])
    o_ = _sigmoid(g[:, 96:128])
    c = f_ * c + i_ * gg
    h = o_ * jnp.tanh(c)
    return h, c


def _node_s2s(feat, wihT, whhT, bih, bhh):
    def body(feat_ref, wih, whh, bihr, bhhr, q_ref):
        feat_v = feat_ref[...]
        h = jnp.zeros((1, 32), jnp.float32)
        c = jnp.zeros((1, 32), jnp.float32)
        q = jnp.zeros((1, 64), jnp.float32)
        for _ in range(2):
            h, c = _lstm_step(q, h, c, wih, whh, bihr, bhhr)
            logits = jnp.sum(feat_v * h, axis=1, keepdims=True)
            m = jnp.max(logits, axis=0, keepdims=True)
            a = jnp.exp(logits - m)
            z = jnp.sum(a, axis=0, keepdims=True)
            r = jnp.sum((a / z) * feat_v, axis=0, keepdims=True)
            q = jnp.concatenate([h, r], axis=1)
        q_ref[...] = q

    return pl.pallas_call(
        body, out_shape=jax.ShapeDtypeStruct((1, 64), jnp.float32),
    )(feat, wihT, whhT, bih, bhh)


def _edge_s2s_pass(feat, wihT, whhT, bih, bhh, blk1T, blk1, fold, prev):
    first = prev is None

    def body(feat_ref, wih, whh, bihr, bhhr, b1t, b1_, fold_r, *rest):
        if first:
            m_ref, z_ref, r_ref, h_ref, c_ref = rest
        else:
            m1, z1, r1p, h1, c1, m_ref, z_ref, r_ref, h_ref = rest
        i = pl.program_id(0)

        @pl.when(i == 0)
        def _():
            if first:
                q = jnp.zeros((1, 64), jnp.float32)
                hp = jnp.zeros((1, 32), jnp.float32)
                cp = jnp.zeros((1, 32), jnp.float32)
            else:
                r1 = (jnp.dot(r1p[...], fold_r[...],
                              preferred_element_type=jnp.float32) / z1[...])
                q = jnp.concatenate([h1[...], r1], axis=1)
                hp = h1[...]
                cp = c1[...]
            h, c = _lstm_step(q, hp, cp, wih, whh, bihr, bhhr)
            h_ref[...] = h
            if first:
                c_ref[...] = c
            m_ref[...] = jnp.full((1, 1), -1e30, jnp.float32)
            z_ref[...] = jnp.zeros((1, 1), jnp.float32)
            r_ref[...] = jnp.zeros((1, 128), jnp.float32)

        h = h_ref[...]
        feat_t = feat_ref[...]
        h_tile = jnp.concatenate([h, h, h, h], axis=1)
        logits = jnp.dot(feat_t * h_tile, b1t[...],
                         preferred_element_type=jnp.float32)
        mt = jnp.max(jnp.max(logits, axis=0, keepdims=True), axis=1,
                     keepdims=True)
        m_old = m_ref[...]
        m_new = jnp.maximum(m_old, mt)
        sc = jnp.exp(m_old - m_new)
        a = jnp.exp(logits - m_new)
        z_ref[...] = (z_ref[...] * sc
                      + jnp.sum(jnp.sum(a, axis=0, keepdims=True), axis=1,
                                keepdims=True))
        ab = jnp.dot(a, b1_[...], preferred_element_type=jnp.float32)
        r_ref[...] = (r_ref[...] * sc
                      + jnp.sum(ab * feat_t, axis=0, keepdims=True))
        m_ref[...] = m_new

    tile = pl.BlockSpec((_TS, 128), lambda i: (i, 0))
    small = [_full((1, 1)), _full((1, 1)), _full((1, 128)), _full((1, 32))]
    in_specs = [tile, _full((64, 128)), _full((32, 128)), _full((1, 128)),
                _full((1, 128)), _full((128, 4)), _full((4, 128)),
                _full((128, 32))]
    out_shape = [jax.ShapeDtypeStruct((1, 1), jnp.float32),
                 jax.ShapeDtypeStruct((1, 1), jnp.float32),
                 jax.ShapeDtypeStruct((1, 128), jnp.float32),
                 jax.ShapeDtypeStruct((1, 32), jnp.float32)]
    out_specs = small[:]
    args = [feat, wihT, whhT, bih, bhh, blk1T, blk1, fold]
    if first:
        out_shape.append(jax.ShapeDtypeStruct((1, 32), jnp.float32))
        out_specs.append(_full((1, 32)))
    else:
        in_specs += small + [_full((1, 32))]
        args += list(prev)
    return pl.pallas_call(
        body, grid=(PE // _TS,), in_specs=in_specs, out_specs=out_specs,
        out_shape=out_shape,
    )(*args)


_MESH = dict(core_axis_name="c", subcore_axis_name="s")


def _sc_gather(vtab, src2, dst2):
    mesh = plsc.VectorSubcoreMesh(**_MESH)

    @functools.partial(
        pl.kernel, mesh=mesh,
        compiler_params=pltpu.CompilerParams(use_tc_tiling_on_sc=False),
        out_type=[jax.ShapeDtypeStruct((E_TOTAL, 32), jnp.float32),
                  jax.ShapeDtypeStruct((E_TOTAL, 32), jnp.float32)],
        scratch_types=[pltpu.VMEM((_GPW, _GRP), jnp.int32),
                       pltpu.VMEM((_GPW, _GRP), jnp.int32),
                       pltpu.VMEM((_CH_E, 32), jnp.float32),
                       pltpu.VMEM((_CH_E, 32), jnp.float32),
                       pltpu.SemaphoreType.DMA],
    )
    def k(vtab_ref, src_ref, dst_ref, vs_ref, vd_ref, idx_s, idx_d,
          buf_s, buf_d, sem):
        tab = vtab_ref
        vs32 = vs_ref
        vd32 = vd_ref
        wid = lax.axis_index("s") * 2 + lax.axis_index("c")
        pltpu.sync_copy(src_ref.at[wid], idx_s)
        pltpu.sync_copy(dst_ref.at[wid], idx_d)
        ebase = wid * _PER_W

        def chunk(ci, carry):
            cps = []
            for j in range(_CGRP):
                g = ci * _CGRP + j
                cps.append(pltpu.async_copy(
                    tab.at[idx_s.at[g]],
                    buf_s.at[pl.ds(j * _GRP, _GRP)], sem))
                cps.append(pltpu.async_copy(
                    tab.at[idx_d.at[g]],
                    buf_d.at[pl.ds(j * _GRP, _GRP)], sem))
            for cp in cps:
                cp.wait()
            ob = ebase + ci * _CH_E
            pltpu.sync_copy(buf_s, vs32.at[pl.ds(ob, _CH_E)])
            pltpu.sync_copy(buf_d, vd32.at[pl.ds(ob, _CH_E)])
            return carry

        lax.fori_loop(0, _NCH, chunk, 0)

    return k(vtab, src2, dst2)


def _sc_scatter_add(enew, dst2, zeros32):
    mesh = plsc.VectorSubcoreMesh(**_MESH)

    @functools.partial(
        pl.kernel, mesh=mesh,
        compiler_params=pltpu.CompilerParams(use_tc_tiling_on_sc=False),
        out_type=jax.ShapeDtypeStruct((2 * N_PAD, 32), jnp.float32),
        scratch_types=[pltpu.VMEM((_GPW, _GRP), jnp.int32),
                       pltpu.VMEM((_CH_E, 32), jnp.float32),
                       pltpu.VMEM_SHARED((N_PAD, 32), jnp.float32),
                       pltpu.SemaphoreType.DMA],
    )
    def k(enew_ref, dst_ref, z_ref, out_ref, idx_d, rows, accum, sem):
        e32 = enew_ref
        o32 = out_ref
        cid = lax.axis_index("c")
        sid = lax.axis_index("s")
        wid = sid * 2 + cid
        pltpu.sync_copy(dst_ref.at[wid], idx_d)
        pltpu.sync_copy(z_ref.at[pl.ds(sid * _NPT, _NPT)],
                        accum.at[pl.ds(sid * _NPT, _NPT)])
        plsc.subcore_barrier()
        ebase = wid * _PER_W

        def chunk(ci, carry):
            pltpu.sync_copy(e32.at[pl.ds(ebase + ci * _CH_E, _CH_E)], rows)
            cps = [pltpu.async_copy(rows.at[pl.ds(j * _GRP, _GRP)],
                                    accum.at[idx_d.at[ci * _CGRP + j]],
                                    sem, add=True)
                   for j in range(_CGRP)]
            for cp in cps:
                cp.wait()
            return carry

        lax.fori_loop(0, _NCH, chunk, 0)
        plsc.subcore_barrier()
        pltpu.sync_copy(accum.at[pl.ds(sid * _NPT, _NPT)],
                        o32.at[pl.ds(cid * N_PAD + sid * _NPT, _NPT)])

    return k(enew, dst2, zeros32)


def _sc_degree(dst2, ones16, zeros16):
    mesh = plsc.VectorSubcoreMesh(**_MESH)

    @functools.partial(
        pl.kernel, mesh=mesh,
        compiler_params=pltpu.CompilerParams(use_tc_tiling_on_sc=False),
        out_type=jax.ShapeDtypeStruct((2 * N_PAD, 16), jnp.float32),
        scratch_types=[pltpu.VMEM((_GPW, _GRP), jnp.int32),
                       pltpu.VMEM((_GRP, 16), jnp.float32),
                       pltpu.VMEM_SHARED((N_PAD, 16), jnp.float32),
                       pltpu.SemaphoreType.DMA],
    )
    def k(dst_ref, ones_ref, z_ref, out_ref, idx_d, onesbuf, accum, sem):
        cid = lax.axis_index("c")
        sid = lax.axis_index("s")
        wid = sid * 2 + cid
        pltpu.sync_copy(dst_ref.at[wid], idx_d)
        pltpu.sync_copy(ones_ref, onesbuf)
        pltpu.sync_copy(z_ref.at[pl.ds(sid * _NPT, _NPT)],
                        accum.at[pl.ds(sid * _NPT, _NPT)])
        plsc.subcore_barrier()

        def chunk(ci, carry):
            cps = [pltpu.async_copy(onesbuf,
                                    accum.at[idx_d.at[ci * _CGRP + j]],
                                    sem, add=True)
                   for j in range(_CGRP)]
            for cp in cps:
                cp.wait()
            return carry

        lax.fori_loop(0, _NCH, chunk, 0)
        plsc.subcore_barrier()
        pltpu.sync_copy(accum.at[pl.ds(sid * _NPT, _NPT)],
                        out_ref.at[pl.ds(cid * N_PAD + sid * _NPT, _NPT)])

    return k(dst2, ones16, zeros16).reshape(2, N_PAD, 16)


def kernel(edge_index, edge_feat, node_feat, state_feat, params):
    def _perm(x):
        return jnp.transpose(x.reshape(_NGRID, 4, _TP), (0, 2, 1)
                             ).reshape(_NW, _GPW, _GRP)

    src2 = _perm(edge_index[0])
    dst2 = _perm(edge_index[1])
    nf2 = node_feat.reshape(N_NODES, 1)
    fold = jnp.kron(jnp.ones((4, 1), jnp.float32),
                    jnp.eye(32, dtype=jnp.float32))
    zeros32 = jnp.zeros((N_PAD, 32), jnp.float32)
    zeros16 = jnp.zeros((N_PAD, 16), jnp.float32)
    ones16 = jnp.ones((_GRP, 16), jnp.float32)

    p = params
    (eW0, eb0), (eW1, eb1) = p['edge_enc']
    e = _edge_encoder(edge_feat, eW0, eb0, eW1, eb1)
    (nW0, nb0), (nW1, nb1) = p['node_enc']
    (sW0, sb0), (sW1, sb1) = p['state_enc']
    v, u = _node_state_encoder(nf2, p['node_emb'], nW0, nb0, nW1, nb1,
                                     state_feat, sW0, sb0, sW1, sb1)

    deg2 = _sc_degree(dst2, ones16, zeros16)

    for b, bp in enumerate(p['blocks']):
        W1, b1 = bp['conv_edge'][0]
        W1_vs, W1_vd, W1_e, W1_u = W1[0:32], W1[32:64], W1[64:96], W1[96:128]
        corr = (jnp.sum(W1_e, axis=0).reshape(1, -1) if b > 0
                else jnp.zeros((1, 64), jnp.float32))
        if b == 0:
            vp, up = v, u
            b1_eff = _bias_eff(u, W1_u, b1, corr)
        else:
            vp, up, b1_eff = _node_prep(v, u, bp['node_func'],
                                        bp['state_func'], W1_u, b1, corr)
        vs_g, vd_g = _sc_gather(vp, src2, dst2)
        (W2, b2), (W3, b3) = bp['conv_edge'][1], bp['conv_edge'][2]
        enew, eout, sume = _edge_update(
            e, vs_g.reshape(PE, 128), vd_g.reshape(PE, 128),
            bp['edge_func'] if b > 0 else None,
            W1_vs, W1_vd, W1_e, b1_eff, W2, b2, W3, b3)
        Sp = _sc_scatter_add(enew.reshape(E_TOTAL, 32), dst2,
                             zeros32).reshape(2, N_PAD, 32)
        v, u = _node_update(v, vp, Sp, deg2, up, sume, fold, u,
                            bp['conv_node'], bp['conv_state'])
        e = eout

    ns = p['node_s2s']
    q_n = _node_s2s(v, ns['W_ih'].T, ns['W_hh'].T,
                    ns['b_ih'].reshape(1, -1), ns['b_hh'].reshape(1, -1))
    es = p['edge_s2s']
    wihT, whhT = es['W_ih'].T, es['W_hh'].T
    bih, bhh = es['b_ih'].reshape(1, -1), es['b_hh'].reshape(1, -1)
    blk1 = jnp.kron(jnp.eye(4, dtype=jnp.float32),
                    jnp.ones((1, 32), jnp.float32))
    blk1T = jnp.kron(jnp.eye(4, dtype=jnp.float32),
                     jnp.ones((32, 1), jnp.float32))
    p1 = _edge_s2s_pass(e, wihT, whhT, bih, bhh, blk1T, blk1, fold, None)
    m2, z2, r2p, h2 = _edge_s2s_pass(e, wihT, whhT, bih, bhh, blk1T, blk1,
                                     fold, p1)
    r2 = r2p.reshape(4, 32).sum(axis=0, keepdims=True) / z2
    edge_vec = jnp.concatenate([h2, r2], axis=1).reshape(64)
    return jnp.hstack([q_n.reshape(64), edge_vec, u.reshape(32)])
